# baseline (speedup 1.0000x reference)
import functools

import jax
import jax.numpy as jnp
from jax import lax
from jax.experimental import pallas as pl
from jax.experimental.pallas import tpu as pltpu
from jax.experimental.pallas import tpu_sc as plsc

E = 32
N = 8192
ADJ = 4096
BETA = 0.25
NC, NS = 2, 16
NW = NC * NS
EDGE_COLS = 128

def _mm1_body(x_ref, w_ref, o_ref):
    o_ref[...] = jnp.dot(x_ref[...], w_ref[...],
                         preferred_element_type=jnp.float32)


def _mm1(code, W1):
    M, K = code.shape
    Nout = W1.shape[1]
    blk = 1024
    return pl.pallas_call(
        _mm1_body,
        grid=(M // blk,),
        in_specs=[pl.BlockSpec((blk, K), lambda i: (i, 0)),
                  pl.BlockSpec((K, Nout), lambda i: (0, 0))],
        out_specs=pl.BlockSpec((blk, Nout), lambda i: (i, 0)),
        out_shape=jax.ShapeDtypeStruct((M, Nout), jnp.float32),
    )(code, W1)


def _sc_degree(dst2d):
    rows_pt = dst2d.shape[0] // NW
    mesh = plsc.VectorSubcoreMesh(core_axis_name="c", subcore_axis_name="s")

    @functools.partial(
        pl.kernel, mesh=mesh,
        out_type=jax.ShapeDtypeStruct((NW, N), jnp.float32),
        scratch_types=[pltpu.VMEM((rows_pt, EDGE_COLS), jnp.int32),
                       pltpu.VMEM((N,), jnp.float32)],
        compiler_params=pltpu.CompilerParams(use_tc_tiling_on_sc=False,
                                             needs_layout_passes=False),
    )
    def k(dst_hbm, out_hbm, dstv, hist):
        c = lax.axis_index("c")
        s = lax.axis_index("s")
        wid = c * NS + s
        z16 = jnp.zeros((16,), jnp.float32)

        def zero_body(i, _):
            hist[pl.ds(i * 16, 16)] = z16
            return 0
        lax.fori_loop(0, N // 16, zero_body, 0)

        pltpu.sync_copy(dst_hbm.at[pl.ds(wid * rows_pt, rows_pt)], dstv)
        ones = jnp.ones((16,), jnp.float32)

        def body(r, _):
            for g in range(EDGE_COLS // 16):
                idx = dstv[r, pl.ds(g * 16, 16)]
                plsc.addupdate_scatter(hist, [idx], ones)
            return 0
        lax.fori_loop(0, rows_pt, body, 0)

        pltpu.sync_copy(hist, out_hbm.at[wid])

    return k(dst2d)


def _sc_scatter(hs, src2d, dst2d):
    rows_pt = src2d.shape[0] // NW
    rows_per_sub = N // NS
    mesh = plsc.VectorSubcoreMesh(core_axis_name="c", subcore_axis_name="s")

    @functools.partial(
        pl.kernel, mesh=mesh,
        out_type=jax.ShapeDtypeStruct((NC, N, E), jnp.float32),
        scratch_types=[
            pltpu.VMEM((rows_pt, EDGE_COLS), jnp.int32),
            pltpu.VMEM((rows_pt, EDGE_COLS), jnp.int32),
            pltpu.VMEM((EDGE_COLS, E), jnp.float32),
            pltpu.VMEM((EDGE_COLS, E), jnp.float32),
            pltpu.VMEM_SHARED((N, E), jnp.float32),
            pltpu.SemaphoreType.DMA,
        ],
        compiler_params=pltpu.CompilerParams(use_tc_tiling_on_sc=False),
    )
    def k(hs_hbm, src_hbm, dst_hbm, out_hbm, srcv, dstv, rows, zb, acc, sem):
        c = lax.axis_index("c")
        s = lax.axis_index("s")
        wid = c * NS + s
        z16 = jnp.zeros((16,), jnp.float32)

        def zb_body(i, _):
            zb[i, pl.ds(0, 16)] = z16
            zb[i, pl.ds(16, 16)] = z16
            return 0
        lax.fori_loop(0, EDGE_COLS, zb_body, 0)
        for t in range(rows_per_sub // EDGE_COLS):
            pltpu.sync_copy(zb, acc.at[pl.ds(s * rows_per_sub + t * EDGE_COLS,
                                             EDGE_COLS)])
        pltpu.sync_copy(src_hbm.at[pl.ds(wid * rows_pt, rows_pt)], srcv)
        pltpu.sync_copy(dst_hbm.at[pl.ds(wid * rows_pt, rows_pt)], dstv)
        plsc.subcore_barrier()

        def body(j, _):
            pltpu.async_copy(hs_hbm.at[srcv.at[j]], rows, sem).wait()
            pltpu.sync_copy(rows, acc.at[dstv.at[j]], add=True)
            return 0
        lax.fori_loop(0, rows_pt, body, 0)
        plsc.subcore_barrier()

        pltpu.sync_copy(acc.at[pl.ds(s * rows_per_sub, rows_per_sub)],
                        out_hbm.at[c, pl.ds(s * rows_per_sub, rows_per_sub)])

    return k(hs, src2d, dst2d)


def _prep_body(degp_ref, mm1_ref, dinv_ref, hs1_ref):
    deg = jnp.sum(degp_ref[...], axis=0) + 1.0
    dinv = 1.0 / jnp.sqrt(deg)
    dinv_ref[...] = dinv[:, None]
    hs1_ref[...] = mm1_ref[...] * dinv[:, None]


def _prep(degp, mm1):
    return pl.pallas_call(
        _prep_body,
        out_shape=[jax.ShapeDtypeStruct((N, 1), jnp.float32),
                   jax.ShapeDtypeStruct((N, E), jnp.float32)],
    )(degp, mm1)


def _layer2_body(accp_ref, hs1_ref, dinv_ref, b1_ref, w2_ref, hs2_ref):
    dinv = dinv_ref[...]
    h2 = dinv * (accp_ref[0] + accp_ref[1] + hs1_ref[...]) + b1_ref[...]
    h2 = jnp.maximum(h2, 0.0)
    hs2_ref[...] = jnp.dot(h2, w2_ref[...],
                           preferred_element_type=jnp.float32) * dinv


def _layer2(accp1, hs1, dinv, b1_2d, W2):
    return pl.pallas_call(
        _layer2_body,
        out_shape=jax.ShapeDtypeStruct((N, E), jnp.float32),
    )(accp1, hs1, dinv, b1_2d, W2)


def _codebooks_body(accp_ref, hs2_ref, dinv_ref, b2_ref,
                    ew_ref, ew2_ref, sq_ref, sq2_ref):
    total = (dinv_ref[...] * (accp_ref[0] + accp_ref[1] + hs2_ref[...])
             + b2_ref[...])
    ew = total[:ADJ]
    ew2 = total[ADJ:]
    ew_ref[...] = ew
    ew2_ref[...] = ew2
    sq_ref[...] = jnp.sum(ew ** 2, axis=1)[None, :]
    sq2_ref[...] = jnp.sum(ew2 ** 2, axis=1)[None, :]


def _codebooks(accp2, hs2, dinv, b2_2d):
    return pl.pallas_call(
        _codebooks_body,
        out_shape=[jax.ShapeDtypeStruct((ADJ, E), jnp.float32),
                   jax.ShapeDtypeStruct((ADJ, E), jnp.float32),
                   jax.ShapeDtypeStruct((1, ADJ), jnp.float32),
                   jax.ShapeDtypeStruct((1, ADJ), jnp.float32)],
    )(accp2, hs2, dinv, b2_2d)


def _quant_body(nrows, zf_ref, zf2_ref, ew_ref, ew2_ref, sq_ref, sq2_ref,
                zq_ref, zq2_ref, i1a_ref, i1b_ref, i2_ref, loss_ref):
    i = pl.program_id(0)
    blk = zf_ref.shape[0]
    iota = lax.broadcasted_iota(jnp.int32, (blk, ADJ), 1)
    big = jnp.int32(2 ** 30)
    nt = (((1,), (1,)), ((), ()))
    nn = (((1,), (0,)), ((), ()))

    zfb = zf_ref[...]
    mm = lax.dot_general(zfb, ew_ref[...], nt,
                         preferred_element_type=jnp.float32)
    d = jnp.sum(zfb ** 2, axis=1, keepdims=True) + sq_ref[...] - 2.0 * mm
    m1 = jnp.min(d, axis=1, keepdims=True)
    i1 = jnp.min(jnp.where(d == m1, iota, big), axis=1)
    oh1 = iota == i1[:, None]
    d2 = jnp.where(oh1, jnp.float32(jnp.inf), d)
    m2 = jnp.min(d2, axis=1, keepdims=True)
    i1b = jnp.min(jnp.where(d2 == m2, iota, big), axis=1)
    ohsum = (oh1 | (iota == i1b[:, None])).astype(jnp.float32)
    g = lax.dot_general(ohsum, ew_ref[...], nn,
                        preferred_element_type=jnp.float32)
    zq = g * 0.5
    zq_ref[...] = zfb + (zq - zfb)
    i1a_ref[...] = i1[:, None]
    i1b_ref[...] = i1b[:, None]

    zf2b = zf2_ref[...]
    mm2 = lax.dot_general(zf2b, ew2_ref[...], nt,
                          preferred_element_type=jnp.float32)
    dn = (jnp.sum(zf2b ** 2, axis=1, keepdims=True) + sq2_ref[...]
          - 2.0 * mm2)
    mn = jnp.min(dn, axis=1, keepdims=True)
    i2 = jnp.min(jnp.where(dn == mn, iota, big), axis=1)
    ohn = (iota == i2[:, None]).astype(jnp.float32)
    zq2 = lax.dot_general(ohn, ew2_ref[...], nn,
                          preferred_element_type=jnp.float32)
    zq2_ref[...] = zf2b + (zq2 - zf2b)
    i2_ref[...] = i2[:, None]

    part = jnp.sum((zq - zfb) ** 2) + jnp.sum((zq2 - zf2b) ** 2)
    contrib = part * ((1.0 + BETA) / (nrows * E))
    prev = jnp.where(i == 0, jnp.zeros((1, 1), jnp.float32), loss_ref[...])
    loss_ref[...] = prev + contrib


def _quant(zf, zf2, ew, ew2, sq, sq2):
    nrows = zf.shape[0]
    blk = 256
    grid = (nrows // blk,)
    full = lambda i: (0, 0)
    row = lambda i: (i, 0)
    return pl.pallas_call(
        functools.partial(_quant_body, nrows),
        grid=grid,
        in_specs=[pl.BlockSpec((blk, E), row),
                  pl.BlockSpec((blk, E), row),
                  pl.BlockSpec((ADJ, E), full),
                  pl.BlockSpec((ADJ, E), full),
                  pl.BlockSpec((1, ADJ), full),
                  pl.BlockSpec((1, ADJ), full)],
        out_specs=[pl.BlockSpec((blk, E), row),
                   pl.BlockSpec((blk, E), row),
                   pl.BlockSpec((blk, 1), row),
                   pl.BlockSpec((blk, 1), row),
                   pl.BlockSpec((blk, 1), row),
                   pl.BlockSpec((1, 1), full)],
        out_shape=[jax.ShapeDtypeStruct((nrows, E), jnp.float32),
                   jax.ShapeDtypeStruct((nrows, E), jnp.float32),
                   jax.ShapeDtypeStruct((nrows, 1), jnp.int32),
                   jax.ShapeDtypeStruct((nrows, 1), jnp.int32),
                   jax.ShapeDtypeStruct((nrows, 1), jnp.int32),
                   jax.ShapeDtypeStruct((1, 1), jnp.float32)],
    )(zf, zf2, ew, ew2, sq, sq2)


def kernel(z, code, edge_index, W1, b1, W2, b2):
    b = z.shape[0]
    src2d = edge_index[0].reshape(-1, EDGE_COLS)
    dst2d = edge_index[1].reshape(-1, EDGE_COLS)

    degp = _sc_degree(dst2d)
    mm1 = _mm1(code, W1)
    dinv, hs1 = _prep(degp, mm1)
    accp1 = _sc_scatter(hs1, src2d, dst2d)
    hs2 = _layer2(accp1, hs1, dinv, b1.reshape(1, E), W2)
    accp2 = _sc_scatter(hs2, src2d, dst2d)
    ew, ew2, sq, sq2 = _codebooks(accp2, hs2, dinv, b2.reshape(1, E))

    zf = jnp.transpose(z[:, :E], (0, 2, 3, 1)).reshape(-1, E)
    zf2 = jnp.transpose(z[:, E:], (0, 2, 3, 1)).reshape(-1, E)
    zq, zq2, i1a, i1b, i2, lossm = _quant(zf, zf2, ew, ew2, sq, sq2)

    h, w = z.shape[2], z.shape[3]
    z_adj_q = jnp.transpose(zq.reshape(b, h, w, E), (0, 3, 1, 2))
    z_noun_q = jnp.transpose(zq2.reshape(b, h, w, E), (0, 3, 1, 2))
    z_q = jnp.concatenate([z_adj_q, z_noun_q], axis=1)
    idx1 = jnp.concatenate([i1a, i1b], axis=1).reshape(b, -1)
    idx2 = i2.reshape(b, -1)
    loss = lossm.reshape(())
    return z_q, loss, idx1, idx2


# speedup vs baseline: 1.1449x; 1.1449x over previous
import functools

import jax
import jax.numpy as jnp
from jax import lax
from jax.experimental import pallas as pl
from jax.experimental.pallas import tpu as pltpu
from jax.experimental.pallas import tpu_sc as plsc

E = 32
N = 8192
ADJ = 4096
BETA = 0.25
NC, NS = 2, 16
NW = NC * NS
EDGE_COLS = 128

def _mm1_body(x_ref, w_ref, o_ref):
    o_ref[...] = jnp.dot(x_ref[...], w_ref[...],
                         preferred_element_type=jnp.float32)


def _mm1(code, W1):
    M, K = code.shape
    Nout = W1.shape[1]
    blk = 1024
    return pl.pallas_call(
        _mm1_body,
        grid=(M // blk,),
        in_specs=[pl.BlockSpec((blk, K), lambda i: (i, 0)),
                  pl.BlockSpec((K, Nout), lambda i: (0, 0))],
        out_specs=pl.BlockSpec((blk, Nout), lambda i: (i, 0)),
        out_shape=jax.ShapeDtypeStruct((M, Nout), jnp.float32),
    )(code, W1)


def _sc_degree(dst2d):
    rows_pt = dst2d.shape[0] // NW
    mesh = plsc.VectorSubcoreMesh(core_axis_name="c", subcore_axis_name="s")

    @functools.partial(
        pl.kernel, mesh=mesh,
        out_type=jax.ShapeDtypeStruct((NW, N), jnp.float32),
        scratch_types=[pltpu.VMEM((rows_pt, EDGE_COLS), jnp.int32),
                       pltpu.VMEM((N,), jnp.float32)],
        compiler_params=pltpu.CompilerParams(use_tc_tiling_on_sc=False,
                                             needs_layout_passes=False),
    )
    def k(dst_hbm, out_hbm, dstv, hist):
        c = lax.axis_index("c")
        s = lax.axis_index("s")
        wid = c * NS + s
        z16 = jnp.zeros((16,), jnp.float32)

        def zero_body(i, _):
            hist[pl.ds(i * 16, 16)] = z16
            return 0
        lax.fori_loop(0, N // 16, zero_body, 0)

        pltpu.sync_copy(dst_hbm.at[pl.ds(wid * rows_pt, rows_pt)], dstv)
        ones = jnp.ones((16,), jnp.float32)

        def body(r, _):
            for g in range(EDGE_COLS // 16):
                idx = dstv[r, pl.ds(g * 16, 16)]
                plsc.addupdate_scatter(hist, [idx], ones)
            return 0
        lax.fori_loop(0, rows_pt, body, 0)

        pltpu.sync_copy(hist, out_hbm.at[wid])

    return k(dst2d)


def _sc_scatter(hs, src2d, dst2d):
    rows_pt = src2d.shape[0] // NW
    rows_per_sub = N // NS
    mesh = plsc.VectorSubcoreMesh(core_axis_name="c", subcore_axis_name="s")

    @functools.partial(
        pl.kernel, mesh=mesh,
        out_type=jax.ShapeDtypeStruct((NC, N, E), jnp.float32),
        scratch_types=[
            pltpu.VMEM((rows_pt, EDGE_COLS), jnp.int32),
            pltpu.VMEM((rows_pt, EDGE_COLS), jnp.int32),
            pltpu.VMEM((EDGE_COLS, E), jnp.float32),
            pltpu.VMEM((EDGE_COLS, E), jnp.float32),
            pltpu.VMEM_SHARED((N, E), jnp.float32),
            pltpu.SemaphoreType.DMA,
        ],
        compiler_params=pltpu.CompilerParams(use_tc_tiling_on_sc=False),
    )
    def k(hs_hbm, src_hbm, dst_hbm, out_hbm, srcv, dstv, rows, zb, acc, sem):
        c = lax.axis_index("c")
        s = lax.axis_index("s")
        wid = c * NS + s
        z16 = jnp.zeros((16,), jnp.float32)

        def zb_body(i, _):
            zb[i, pl.ds(0, 16)] = z16
            zb[i, pl.ds(16, 16)] = z16
            return 0
        lax.fori_loop(0, EDGE_COLS, zb_body, 0)
        for t in range(rows_per_sub // EDGE_COLS):
            pltpu.sync_copy(zb, acc.at[pl.ds(s * rows_per_sub + t * EDGE_COLS,
                                             EDGE_COLS)])
        pltpu.sync_copy(src_hbm.at[pl.ds(wid * rows_pt, rows_pt)], srcv)
        pltpu.sync_copy(dst_hbm.at[pl.ds(wid * rows_pt, rows_pt)], dstv)
        plsc.subcore_barrier()

        def body(j, _):
            pltpu.async_copy(hs_hbm.at[srcv.at[j]], rows, sem).wait()
            pltpu.sync_copy(rows, acc.at[dstv.at[j]], add=True)
            return 0
        lax.fori_loop(0, rows_pt, body, 0)
        plsc.subcore_barrier()

        pltpu.sync_copy(acc.at[pl.ds(s * rows_per_sub, rows_per_sub)],
                        out_hbm.at[c, pl.ds(s * rows_per_sub, rows_per_sub)])

    return k(hs, src2d, dst2d)


def _prep_body(degp_ref, mm1_ref, dinv_ref, hs1_ref):
    deg = jnp.sum(degp_ref[...], axis=0) + 1.0
    dinv = 1.0 / jnp.sqrt(deg)
    dinv_ref[...] = dinv[:, None]
    hs1_ref[...] = mm1_ref[...] * dinv[:, None]


def _prep(degp, mm1):
    return pl.pallas_call(
        _prep_body,
        out_shape=[jax.ShapeDtypeStruct((N, 1), jnp.float32),
                   jax.ShapeDtypeStruct((N, E), jnp.float32)],
    )(degp, mm1)


def _layer2_body(accp_ref, hs1_ref, dinv_ref, b1_ref, w2_ref, hs2_ref):
    dinv = dinv_ref[...]
    h2 = dinv * (accp_ref[0] + accp_ref[1] + hs1_ref[...]) + b1_ref[...]
    h2 = jnp.maximum(h2, 0.0)
    hs2_ref[...] = jnp.dot(h2, w2_ref[...],
                           preferred_element_type=jnp.float32) * dinv


def _layer2(accp1, hs1, dinv, b1_2d, W2):
    return pl.pallas_call(
        _layer2_body,
        out_shape=jax.ShapeDtypeStruct((N, E), jnp.float32),
    )(accp1, hs1, dinv, b1_2d, W2)


def _codebooks_body(accp_ref, hs2_ref, dinv_ref, b2_ref,
                    ew_ref, ew2_ref, ewt_ref, sqall_ref):
    total = (dinv_ref[...] * (accp_ref[0] + accp_ref[1] + hs2_ref[...])
             + b2_ref[...])
    ew = total[:ADJ]
    ew2 = total[ADJ:]
    ew_ref[...] = ew
    ew2_ref[...] = ew2
    zpad = jnp.zeros((E, ADJ), jnp.float32)
    top = jnp.concatenate([ew.T, zpad], axis=1)
    bot = jnp.concatenate([zpad, ew2.T], axis=1)
    ewt_ref[...] = jnp.concatenate([top, bot], axis=0)
    sqall_ref[...] = jnp.concatenate(
        [jnp.sum(ew ** 2, axis=1), jnp.sum(ew2 ** 2, axis=1)])[None, :]


def _codebooks(accp2, hs2, dinv, b2_2d):
    return pl.pallas_call(
        _codebooks_body,
        out_shape=[jax.ShapeDtypeStruct((ADJ, E), jnp.float32),
                   jax.ShapeDtypeStruct((ADJ, E), jnp.float32),
                   jax.ShapeDtypeStruct((2 * E, 2 * ADJ), jnp.float32),
                   jax.ShapeDtypeStruct((1, 2 * ADJ), jnp.float32)],
    )(accp2, hs2, dinv, b2_2d)


def _quant_body(nrows, zf_ref, zf2_ref, ew_ref, ew2_ref, ewt_ref, sqall_ref,
                zq_ref, zq2_ref, i1a_ref, i1b_ref, i2_ref, loss_ref):
    i = pl.program_id(0)
    blk = zf_ref.shape[0]
    iota = lax.broadcasted_iota(jnp.int32, (blk, ADJ), 1)
    big = jnp.int32(2 ** 30)
    nn = (((1,), (0,)), ((), ()))

    zfb = zf_ref[...]
    zf2b = zf2_ref[...]
    zcat = jnp.concatenate([zfb, zf2b], axis=1)
    mmall = lax.dot_general(zcat, ewt_ref[...], nn,
                            preferred_element_type=jnp.float32)
    sqall = sqall_ref[...]

    d = (jnp.sum(zfb ** 2, axis=1, keepdims=True) + sqall[:, :ADJ]
         - 2.0 * mmall[:, :ADJ])
    m1 = jnp.min(d, axis=1, keepdims=True)
    i1 = jnp.min(jnp.where(d == m1, iota, big), axis=1)
    oh1 = iota == i1[:, None]
    d2 = jnp.where(oh1, jnp.float32(jnp.inf), d)
    m2 = jnp.min(d2, axis=1, keepdims=True)
    i1b = jnp.min(jnp.where(d2 == m2, iota, big), axis=1)
    ohsum = (oh1 | (iota == i1b[:, None])).astype(jnp.float32)
    g = lax.dot_general(ohsum, ew_ref[...], nn,
                        preferred_element_type=jnp.float32)
    zq = g * 0.5
    zq_ref[...] = zfb + (zq - zfb)
    i1a_ref[...] = i1[:, None]
    i1b_ref[...] = i1b[:, None]

    dn = (jnp.sum(zf2b ** 2, axis=1, keepdims=True) + sqall[:, ADJ:]
          - 2.0 * mmall[:, ADJ:])
    mn = jnp.min(dn, axis=1, keepdims=True)
    i2 = jnp.min(jnp.where(dn == mn, iota, big), axis=1)
    ohn = (iota == i2[:, None]).astype(jnp.float32)
    zq2 = lax.dot_general(ohn, ew2_ref[...], nn,
                          preferred_element_type=jnp.float32)
    zq2_ref[...] = zf2b + (zq2 - zf2b)
    i2_ref[...] = i2[:, None]

    part = jnp.sum((zq - zfb) ** 2) + jnp.sum((zq2 - zf2b) ** 2)
    contrib = part * ((1.0 + BETA) / (nrows * E))
    prev = jnp.where(i == 0, jnp.zeros((1, 1), jnp.float32), loss_ref[...])
    loss_ref[...] = prev + contrib


def _quant(zf, zf2, ew, ew2, ewt, sqall):
    nrows = zf.shape[0]
    blk = 256
    grid = (nrows // blk,)
    full = lambda i: (0, 0)
    row = lambda i: (i, 0)
    return pl.pallas_call(
        functools.partial(_quant_body, nrows),
        grid=grid,
        in_specs=[pl.BlockSpec((blk, E), row),
                  pl.BlockSpec((blk, E), row),
                  pl.BlockSpec((ADJ, E), full),
                  pl.BlockSpec((ADJ, E), full),
                  pl.BlockSpec((2 * E, 2 * ADJ), full),
                  pl.BlockSpec((1, 2 * ADJ), full)],
        out_specs=[pl.BlockSpec((blk, E), row),
                   pl.BlockSpec((blk, E), row),
                   pl.BlockSpec((blk, 1), row),
                   pl.BlockSpec((blk, 1), row),
                   pl.BlockSpec((blk, 1), row),
                   pl.BlockSpec((1, 1), full)],
        out_shape=[jax.ShapeDtypeStruct((nrows, E), jnp.float32),
                   jax.ShapeDtypeStruct((nrows, E), jnp.float32),
                   jax.ShapeDtypeStruct((nrows, 1), jnp.int32),
                   jax.ShapeDtypeStruct((nrows, 1), jnp.int32),
                   jax.ShapeDtypeStruct((nrows, 1), jnp.int32),
                   jax.ShapeDtypeStruct((1, 1), jnp.float32)],
    )(zf, zf2, ew, ew2, ewt, sqall)


def kernel(z, code, edge_index, W1, b1, W2, b2):
    b = z.shape[0]
    src2d = edge_index[0].reshape(-1, EDGE_COLS)
    dst2d = edge_index[1].reshape(-1, EDGE_COLS)

    degp = _sc_degree(dst2d)
    mm1 = _mm1(code, W1)
    dinv, hs1 = _prep(degp, mm1)
    accp1 = _sc_scatter(hs1, src2d, dst2d)
    hs2 = _layer2(accp1, hs1, dinv, b1.reshape(1, E), W2)
    accp2 = _sc_scatter(hs2, src2d, dst2d)
    ew, ew2, ewt, sqall = _codebooks(accp2, hs2, dinv, b2.reshape(1, E))

    zf = jnp.transpose(z[:, :E], (0, 2, 3, 1)).reshape(-1, E)
    zf2 = jnp.transpose(z[:, E:], (0, 2, 3, 1)).reshape(-1, E)
    zq, zq2, i1a, i1b, i2, lossm = _quant(zf, zf2, ew, ew2, ewt, sqall)

    h, w = z.shape[2], z.shape[3]
    z_adj_q = jnp.transpose(zq.reshape(b, h, w, E), (0, 3, 1, 2))
    z_noun_q = jnp.transpose(zq2.reshape(b, h, w, E), (0, 3, 1, 2))
    z_q = jnp.concatenate([z_adj_q, z_noun_q], axis=1)
    idx1 = jnp.concatenate([i1a, i1b], axis=1).reshape(b, -1)
    idx2 = i2.reshape(b, -1)
    loss = lossm.reshape(())
    return z_q, loss, idx1, idx2


# speedup vs baseline: 1.2104x; 1.0572x over previous
import functools

import jax
import jax.numpy as jnp
from jax import lax
from jax.experimental import pallas as pl
from jax.experimental.pallas import tpu as pltpu
from jax.experimental.pallas import tpu_sc as plsc

E = 32
N = 8192
ADJ = 4096
BETA = 0.25
NC, NS = 2, 16
NW = NC * NS
EDGE_COLS = 128

def _mm1_body(x_ref, w_ref, o_ref):
    o_ref[...] = jnp.dot(x_ref[...], w_ref[...],
                         preferred_element_type=jnp.float32)


def _mm1(code, W1):
    M, K = code.shape
    Nout = W1.shape[1]
    blk = 1024
    return pl.pallas_call(
        _mm1_body,
        grid=(M // blk,),
        in_specs=[pl.BlockSpec((blk, K), lambda i: (i, 0)),
                  pl.BlockSpec((K, Nout), lambda i: (0, 0))],
        out_specs=pl.BlockSpec((blk, Nout), lambda i: (i, 0)),
        out_shape=jax.ShapeDtypeStruct((M, Nout), jnp.float32),
    )(code, W1)


def _sc_degree(dst2d):
    rows_pt = dst2d.shape[0] // NW
    mesh = plsc.VectorSubcoreMesh(core_axis_name="c", subcore_axis_name="s")

    @functools.partial(
        pl.kernel, mesh=mesh,
        out_type=jax.ShapeDtypeStruct((NW, N), jnp.float32),
        scratch_types=[pltpu.VMEM((rows_pt, EDGE_COLS), jnp.int32),
                       pltpu.VMEM((N,), jnp.float32)],
        compiler_params=pltpu.CompilerParams(use_tc_tiling_on_sc=False,
                                             needs_layout_passes=False),
    )
    def k(dst_hbm, out_hbm, dstv, hist):
        c = lax.axis_index("c")
        s = lax.axis_index("s")
        wid = c * NS + s
        z16 = jnp.zeros((16,), jnp.float32)

        def zero_body(i, _):
            hist[pl.ds(i * 16, 16)] = z16
            return 0
        lax.fori_loop(0, N // 16, zero_body, 0)

        pltpu.sync_copy(dst_hbm.at[pl.ds(wid * rows_pt, rows_pt)], dstv)
        ones = jnp.ones((16,), jnp.float32)

        def body(r, _):
            for g in range(EDGE_COLS // 16):
                idx = dstv[r, pl.ds(g * 16, 16)]
                plsc.addupdate_scatter(hist, [idx], ones)
            return 0
        lax.fori_loop(0, rows_pt, body, 0)

        pltpu.sync_copy(hist, out_hbm.at[wid])

    return k(dst2d)


def _sc_scatter(hs, src2d, dst2d):
    rows_pt = src2d.shape[0] // NW
    rows_per_sub = N // NS
    mesh = plsc.VectorSubcoreMesh(core_axis_name="c", subcore_axis_name="s")

    @functools.partial(
        pl.kernel, mesh=mesh,
        out_type=jax.ShapeDtypeStruct((NC, N, E), jnp.float32),
        scratch_types=[
            pltpu.VMEM((rows_pt, EDGE_COLS), jnp.int32),
            pltpu.VMEM((rows_pt, EDGE_COLS), jnp.int32),
            pltpu.VMEM((EDGE_COLS, E), jnp.float32),
            pltpu.VMEM((EDGE_COLS, E), jnp.float32),
            pltpu.VMEM_SHARED((N, E), jnp.float32),
            pltpu.SemaphoreType.DMA,
        ],
        compiler_params=pltpu.CompilerParams(use_tc_tiling_on_sc=False),
    )
    def k(hs_hbm, src_hbm, dst_hbm, out_hbm, srcv, dstv, rows, zb, acc, sem):
        c = lax.axis_index("c")
        s = lax.axis_index("s")
        wid = c * NS + s
        z16 = jnp.zeros((16,), jnp.float32)

        def zb_body(i, _):
            zb[i, pl.ds(0, 16)] = z16
            zb[i, pl.ds(16, 16)] = z16
            return 0
        lax.fori_loop(0, EDGE_COLS, zb_body, 0)
        for t in range(rows_per_sub // EDGE_COLS):
            pltpu.sync_copy(zb, acc.at[pl.ds(s * rows_per_sub + t * EDGE_COLS,
                                             EDGE_COLS)])
        pltpu.sync_copy(src_hbm.at[pl.ds(wid * rows_pt, rows_pt)], srcv)
        pltpu.sync_copy(dst_hbm.at[pl.ds(wid * rows_pt, rows_pt)], dstv)
        plsc.subcore_barrier()

        def body(j, _):
            pltpu.async_copy(hs_hbm.at[srcv.at[j]], rows, sem).wait()
            pltpu.sync_copy(rows, acc.at[dstv.at[j]], add=True)
            return 0
        lax.fori_loop(0, rows_pt, body, 0)
        plsc.subcore_barrier()

        pltpu.sync_copy(acc.at[pl.ds(s * rows_per_sub, rows_per_sub)],
                        out_hbm.at[c, pl.ds(s * rows_per_sub, rows_per_sub)])

    return k(hs, src2d, dst2d)


def _sc_gather(ew, ew2, i1a2d, i1b2d, i22d):
    nq = i1a2d.shape[0] * i1a2d.shape[1]
    rows_pt = i1a2d.shape[0] // NW
    mesh = plsc.VectorSubcoreMesh(core_axis_name="c", subcore_axis_name="s")

    @functools.partial(
        pl.kernel, mesh=mesh,
        out_type=[jax.ShapeDtypeStruct((nq, E), jnp.float32),
                  jax.ShapeDtypeStruct((nq, E), jnp.float32)],
        scratch_types=[
            pltpu.VMEM((rows_pt, EDGE_COLS), jnp.int32),
            pltpu.VMEM((rows_pt, EDGE_COLS), jnp.int32),
            pltpu.VMEM((rows_pt, EDGE_COLS), jnp.int32),
            pltpu.VMEM((EDGE_COLS, E), jnp.float32),
            pltpu.VMEM((EDGE_COLS, E), jnp.float32),
            pltpu.VMEM((EDGE_COLS, E), jnp.float32),
            pltpu.SemaphoreType.DMA,
        ],
        compiler_params=pltpu.CompilerParams(use_tc_tiling_on_sc=False),
    )
    def k(ew_hbm, ew2_hbm, ia_hbm, ib_hbm, i2_hbm, zq_hbm, zq2_hbm,
          iav, ibv, i2v, r1, r2, r3, sem):
        c = lax.axis_index("c")
        s = lax.axis_index("s")
        wid = c * NS + s
        pltpu.sync_copy(ia_hbm.at[pl.ds(wid * rows_pt, rows_pt)], iav)
        pltpu.sync_copy(ib_hbm.at[pl.ds(wid * rows_pt, rows_pt)], ibv)
        pltpu.sync_copy(i2_hbm.at[pl.ds(wid * rows_pt, rows_pt)], i2v)

        def body(j, _):
            base = (wid * rows_pt + j) * EDGE_COLS
            pltpu.async_copy(ew_hbm.at[iav.at[j]], r1, sem).wait()
            pltpu.async_copy(ew_hbm.at[ibv.at[j]], r2, sem).wait()
            pltpu.async_copy(ew2_hbm.at[i2v.at[j]], r3, sem).wait()

            def row_body(r, _):
                for cc in range(E // 16):
                    sl = pl.ds(cc * 16, 16)
                    r1[r, sl] = (r1[r, sl] + r2[r, sl]) * 0.5
                return 0
            lax.fori_loop(0, EDGE_COLS, row_body, 0)
            pltpu.sync_copy(r1, zq_hbm.at[pl.ds(base, EDGE_COLS)])
            pltpu.sync_copy(r3, zq2_hbm.at[pl.ds(base, EDGE_COLS)])
            return 0
        lax.fori_loop(0, rows_pt, body, 0)

    return k(ew, ew2, i1a2d, i1b2d, i22d)


def _finish_body(nrows, zf_ref, zf2_ref, zq_ref, zq2_ref,
                 zqs_ref, zq2s_ref, loss_ref):
    i = pl.program_id(0)
    zfb = zf_ref[...]
    zf2b = zf2_ref[...]
    zq = zq_ref[...]
    zq2 = zq2_ref[...]
    zqs_ref[...] = zfb + (zq - zfb)
    zq2s_ref[...] = zf2b + (zq2 - zf2b)
    part = jnp.sum((zq - zfb) ** 2) + jnp.sum((zq2 - zf2b) ** 2)
    contrib = part * ((1.0 + BETA) / (nrows * E))
    prev = jnp.where(i == 0, jnp.zeros((1, 1), jnp.float32), loss_ref[...])
    loss_ref[...] = prev + contrib


def _finish(zf, zf2, zq, zq2):
    nrows = zf.shape[0]
    blk = 2048
    row = lambda i: (i, 0)
    full = lambda i: (0, 0)
    return pl.pallas_call(
        functools.partial(_finish_body, nrows),
        grid=(nrows // blk,),
        in_specs=[pl.BlockSpec((blk, E), row)] * 4,
        out_specs=[pl.BlockSpec((blk, E), row),
                   pl.BlockSpec((blk, E), row),
                   pl.BlockSpec((1, 1), full)],
        out_shape=[jax.ShapeDtypeStruct((nrows, E), jnp.float32),
                   jax.ShapeDtypeStruct((nrows, E), jnp.float32),
                   jax.ShapeDtypeStruct((1, 1), jnp.float32)],
    )(zf, zf2, zq, zq2)


def _prep_body(degp_ref, mm1_ref, dinv_ref, hs1_ref):
    deg = jnp.sum(degp_ref[...], axis=0) + 1.0
    dinv = 1.0 / jnp.sqrt(deg)
    dinv_ref[...] = dinv[:, None]
    hs1_ref[...] = mm1_ref[...] * dinv[:, None]


def _prep(degp, mm1):
    return pl.pallas_call(
        _prep_body,
        out_shape=[jax.ShapeDtypeStruct((N, 1), jnp.float32),
                   jax.ShapeDtypeStruct((N, E), jnp.float32)],
    )(degp, mm1)


def _layer2_body(accp_ref, hs1_ref, dinv_ref, b1_ref, w2_ref, hs2_ref):
    dinv = dinv_ref[...]
    h2 = dinv * (accp_ref[0] + accp_ref[1] + hs1_ref[...]) + b1_ref[...]
    h2 = jnp.maximum(h2, 0.0)
    hs2_ref[...] = jnp.dot(h2, w2_ref[...],
                           preferred_element_type=jnp.float32) * dinv


def _layer2(accp1, hs1, dinv, b1_2d, W2):
    return pl.pallas_call(
        _layer2_body,
        out_shape=jax.ShapeDtypeStruct((N, E), jnp.float32),
    )(accp1, hs1, dinv, b1_2d, W2)


def _codebooks_body(accp_ref, hs2_ref, dinv_ref, b2_ref,
                    ew_ref, ew2_ref, ewt_ref, sqall_ref):
    total = (dinv_ref[...] * (accp_ref[0] + accp_ref[1] + hs2_ref[...])
             + b2_ref[...])
    ew = total[:ADJ]
    ew2 = total[ADJ:]
    ew_ref[...] = ew
    ew2_ref[...] = ew2
    zpad = jnp.zeros((E, ADJ), jnp.float32)
    top = jnp.concatenate([ew.T, zpad], axis=1)
    bot = jnp.concatenate([zpad, ew2.T], axis=1)
    ewt_ref[...] = jnp.concatenate([top, bot], axis=0)
    sqall_ref[...] = jnp.concatenate(
        [jnp.sum(ew ** 2, axis=1), jnp.sum(ew2 ** 2, axis=1)])[None, :]


def _codebooks(accp2, hs2, dinv, b2_2d):
    return pl.pallas_call(
        _codebooks_body,
        out_shape=[jax.ShapeDtypeStruct((ADJ, E), jnp.float32),
                   jax.ShapeDtypeStruct((ADJ, E), jnp.float32),
                   jax.ShapeDtypeStruct((2 * E, 2 * ADJ), jnp.float32),
                   jax.ShapeDtypeStruct((1, 2 * ADJ), jnp.float32)],
    )(accp2, hs2, dinv, b2_2d)


def _quant_body(zf_ref, zf2_ref, ewt_ref, sqall_ref,
                i1a_ref, i1b_ref, i2_ref):
    blk = zf_ref.shape[0]
    iota = lax.broadcasted_iota(jnp.int32, (blk, ADJ), 1)
    big = jnp.int32(2 ** 30)
    nn = (((1,), (0,)), ((), ()))

    zfb = zf_ref[...]
    zf2b = zf2_ref[...]
    zcat = jnp.concatenate([zfb, zf2b], axis=1)
    mmall = lax.dot_general(zcat, ewt_ref[...], nn,
                            preferred_element_type=jnp.float32)
    sqall = sqall_ref[...]

    d = (jnp.sum(zfb ** 2, axis=1, keepdims=True) + sqall[:, :ADJ]
         - 2.0 * mmall[:, :ADJ])
    m1 = jnp.min(d, axis=1, keepdims=True)
    i1 = jnp.min(jnp.where(d == m1, iota, big), axis=1)
    oh1 = iota == i1[:, None]
    d2 = jnp.where(oh1, jnp.float32(jnp.inf), d)
    m2 = jnp.min(d2, axis=1, keepdims=True)
    i1b = jnp.min(jnp.where(d2 == m2, iota, big), axis=1)
    i1a_ref[...] = i1[:, None]
    i1b_ref[...] = i1b[:, None]

    dn = (jnp.sum(zf2b ** 2, axis=1, keepdims=True) + sqall[:, ADJ:]
          - 2.0 * mmall[:, ADJ:])
    mn = jnp.min(dn, axis=1, keepdims=True)
    i2 = jnp.min(jnp.where(dn == mn, iota, big), axis=1)
    i2_ref[...] = i2[:, None]


def _quant(zf, zf2, ewt, sqall):
    nrows = zf.shape[0]
    blk = 256
    grid = (nrows // blk,)
    full = lambda i: (0, 0)
    row = lambda i: (i, 0)
    return pl.pallas_call(
        _quant_body,
        grid=grid,
        in_specs=[pl.BlockSpec((blk, E), row),
                  pl.BlockSpec((blk, E), row),
                  pl.BlockSpec((2 * E, 2 * ADJ), full),
                  pl.BlockSpec((1, 2 * ADJ), full)],
        out_specs=[pl.BlockSpec((blk, 1), row),
                   pl.BlockSpec((blk, 1), row),
                   pl.BlockSpec((blk, 1), row)],
        out_shape=[jax.ShapeDtypeStruct((nrows, 1), jnp.int32),
                   jax.ShapeDtypeStruct((nrows, 1), jnp.int32),
                   jax.ShapeDtypeStruct((nrows, 1), jnp.int32)],
    )(zf, zf2, ewt, sqall)


def kernel(z, code, edge_index, W1, b1, W2, b2):
    b = z.shape[0]
    src2d = edge_index[0].reshape(-1, EDGE_COLS)
    dst2d = edge_index[1].reshape(-1, EDGE_COLS)

    degp = _sc_degree(dst2d)
    mm1 = _mm1(code, W1)
    dinv, hs1 = _prep(degp, mm1)
    accp1 = _sc_scatter(hs1, src2d, dst2d)
    hs2 = _layer2(accp1, hs1, dinv, b1.reshape(1, E), W2)
    accp2 = _sc_scatter(hs2, src2d, dst2d)
    ew, ew2, ewt, sqall = _codebooks(accp2, hs2, dinv, b2.reshape(1, E))

    zf = jnp.transpose(z[:, :E], (0, 2, 3, 1)).reshape(-1, E)
    zf2 = jnp.transpose(z[:, E:], (0, 2, 3, 1)).reshape(-1, E)
    i1a, i1b, i2 = _quant(zf, zf2, ewt, sqall)
    zqr, zq2r = _sc_gather(ew, ew2,
                           i1a.reshape(-1, EDGE_COLS),
                           i1b.reshape(-1, EDGE_COLS),
                           i2.reshape(-1, EDGE_COLS))
    zq, zq2, lossm = _finish(zf, zf2, zqr, zq2r)

    h, w = z.shape[2], z.shape[3]
    z_adj_q = jnp.transpose(zq.reshape(b, h, w, E), (0, 3, 1, 2))
    z_noun_q = jnp.transpose(zq2.reshape(b, h, w, E), (0, 3, 1, 2))
    z_q = jnp.concatenate([z_adj_q, z_noun_q], axis=1)
    idx1 = jnp.concatenate([i1a, i1b], axis=1).reshape(b, -1)
    idx2 = i2.reshape(b, -1)
    loss = lossm.reshape(())
    return z_q, loss, idx1, idx2


# speedup vs baseline: 1.2971x; 1.0717x over previous
import functools

import jax
import jax.numpy as jnp
from jax import lax
from jax.experimental import pallas as pl
from jax.experimental.pallas import tpu as pltpu
from jax.experimental.pallas import tpu_sc as plsc

E = 32
N = 8192
ADJ = 4096
BETA = 0.25
NC, NS = 2, 16
NW = NC * NS
EDGE_COLS = 128

def _mm1_body(x_ref, w_ref, o_ref):
    o_ref[...] = jnp.dot(x_ref[...], w_ref[...],
                         preferred_element_type=jnp.float32)


def _mm1(code, W1):
    M, K = code.shape
    Nout = W1.shape[1]
    blk = 1024
    return pl.pallas_call(
        _mm1_body,
        grid=(M // blk,),
        in_specs=[pl.BlockSpec((blk, K), lambda i: (i, 0)),
                  pl.BlockSpec((K, Nout), lambda i: (0, 0))],
        out_specs=pl.BlockSpec((blk, Nout), lambda i: (i, 0)),
        out_shape=jax.ShapeDtypeStruct((M, Nout), jnp.float32),
    )(code, W1)


def _sc_degree(dst2d):
    rows_pt = dst2d.shape[0] // NW
    mesh = plsc.VectorSubcoreMesh(core_axis_name="c", subcore_axis_name="s")

    @functools.partial(
        pl.kernel, mesh=mesh,
        out_type=jax.ShapeDtypeStruct((NW, N), jnp.float32),
        scratch_types=[pltpu.VMEM((rows_pt, EDGE_COLS), jnp.int32),
                       pltpu.VMEM((N,), jnp.float32)],
        compiler_params=pltpu.CompilerParams(use_tc_tiling_on_sc=False,
                                             needs_layout_passes=False),
    )
    def k(dst_hbm, out_hbm, dstv, hist):
        c = lax.axis_index("c")
        s = lax.axis_index("s")
        wid = c * NS + s
        z16 = jnp.zeros((16,), jnp.float32)

        def zero_body(i, _):
            hist[pl.ds(i * 16, 16)] = z16
            return 0
        lax.fori_loop(0, N // 16, zero_body, 0)

        pltpu.sync_copy(dst_hbm.at[pl.ds(wid * rows_pt, rows_pt)], dstv)
        ones = jnp.ones((16,), jnp.float32)

        def body(r, _):
            for g in range(EDGE_COLS // 16):
                idx = dstv[r, pl.ds(g * 16, 16)]
                plsc.addupdate_scatter(hist, [idx], ones)
            return 0
        lax.fori_loop(0, rows_pt, body, 0)

        pltpu.sync_copy(hist, out_hbm.at[wid])

    return k(dst2d)


def _sc_scatter(hs, src2d, dst2d):
    rows_pt = src2d.shape[0] // NW
    rows_per_sub = N // NS
    mesh = plsc.VectorSubcoreMesh(core_axis_name="c", subcore_axis_name="s")

    @functools.partial(
        pl.kernel, mesh=mesh,
        out_type=jax.ShapeDtypeStruct((NC, N, E), jnp.float32),
        scratch_types=[
            pltpu.VMEM((rows_pt, EDGE_COLS), jnp.int32),
            pltpu.VMEM((rows_pt, EDGE_COLS), jnp.int32),
            pltpu.VMEM((EDGE_COLS, E), jnp.float32),
            pltpu.VMEM((EDGE_COLS, E), jnp.float32),
            pltpu.VMEM_SHARED((N, E), jnp.float32),
            pltpu.SemaphoreType.DMA,
        ],
        compiler_params=pltpu.CompilerParams(use_tc_tiling_on_sc=False),
    )
    def k(hs_hbm, src_hbm, dst_hbm, out_hbm, srcv, dstv, rows, zb, acc, sem):
        c = lax.axis_index("c")
        s = lax.axis_index("s")
        wid = c * NS + s
        z16 = jnp.zeros((16,), jnp.float32)

        def zb_body(i, _):
            zb[i, pl.ds(0, 16)] = z16
            zb[i, pl.ds(16, 16)] = z16
            return 0
        lax.fori_loop(0, EDGE_COLS, zb_body, 0)
        for t in range(rows_per_sub // EDGE_COLS):
            pltpu.sync_copy(zb, acc.at[pl.ds(s * rows_per_sub + t * EDGE_COLS,
                                             EDGE_COLS)])
        pltpu.sync_copy(src_hbm.at[pl.ds(wid * rows_pt, rows_pt)], srcv)
        pltpu.sync_copy(dst_hbm.at[pl.ds(wid * rows_pt, rows_pt)], dstv)
        plsc.subcore_barrier()

        def body(j, _):
            pltpu.async_copy(hs_hbm.at[srcv.at[j]], rows, sem).wait()
            pltpu.sync_copy(rows, acc.at[dstv.at[j]], add=True)
            return 0
        lax.fori_loop(0, rows_pt, body, 0)
        plsc.subcore_barrier()

        pltpu.sync_copy(acc.at[pl.ds(s * rows_per_sub, rows_per_sub)],
                        out_hbm.at[c, pl.ds(s * rows_per_sub, rows_per_sub)])

    return k(hs, src2d, dst2d)


def _sc_gather(ew, ew2, i1a2d, i1b2d, i22d):
    nq = i1a2d.shape[0] * i1a2d.shape[1]
    rows_pt = i1a2d.shape[0] // NW
    mesh = plsc.VectorSubcoreMesh(core_axis_name="c", subcore_axis_name="s")

    @functools.partial(
        pl.kernel, mesh=mesh,
        out_type=[jax.ShapeDtypeStruct((nq, E), jnp.float32),
                  jax.ShapeDtypeStruct((nq, E), jnp.float32)],
        scratch_types=[
            pltpu.VMEM((rows_pt, EDGE_COLS), jnp.int32),
            pltpu.VMEM((rows_pt, EDGE_COLS), jnp.int32),
            pltpu.VMEM((rows_pt, EDGE_COLS), jnp.int32),
            pltpu.VMEM((EDGE_COLS, E), jnp.float32),
            pltpu.VMEM((EDGE_COLS, E), jnp.float32),
            pltpu.VMEM((EDGE_COLS, E), jnp.float32),
            pltpu.SemaphoreType.DMA,
        ],
        compiler_params=pltpu.CompilerParams(use_tc_tiling_on_sc=False),
    )
    def k(ew_hbm, ew2_hbm, ia_hbm, ib_hbm, i2_hbm, zq_hbm, zq2_hbm,
          iav, ibv, i2v, r1, r2, r3, sem):
        c = lax.axis_index("c")
        s = lax.axis_index("s")
        wid = c * NS + s
        pltpu.sync_copy(ia_hbm.at[pl.ds(wid * rows_pt, rows_pt)], iav)
        pltpu.sync_copy(ib_hbm.at[pl.ds(wid * rows_pt, rows_pt)], ibv)
        pltpu.sync_copy(i2_hbm.at[pl.ds(wid * rows_pt, rows_pt)], i2v)

        def body(j, _):
            base = (wid * rows_pt + j) * EDGE_COLS
            pltpu.async_copy(ew_hbm.at[iav.at[j]], r1, sem).wait()
            pltpu.async_copy(ew_hbm.at[ibv.at[j]], r2, sem).wait()
            pltpu.async_copy(ew2_hbm.at[i2v.at[j]], r3, sem).wait()

            def row_body(r, _):
                for cc in range(E // 16):
                    sl = pl.ds(cc * 16, 16)
                    r1[r, sl] = (r1[r, sl] + r2[r, sl]) * 0.5
                return 0
            lax.fori_loop(0, EDGE_COLS, row_body, 0)
            pltpu.sync_copy(r1, zq_hbm.at[pl.ds(base, EDGE_COLS)])
            pltpu.sync_copy(r3, zq2_hbm.at[pl.ds(base, EDGE_COLS)])
            return 0
        lax.fori_loop(0, rows_pt, body, 0)

    return k(ew, ew2, i1a2d, i1b2d, i22d)


def _finish_body(nrows, zf_ref, zf2_ref, zq_ref, zq2_ref,
                 zqs_ref, zq2s_ref, loss_ref):
    i = pl.program_id(0)
    zfb = zf_ref[...]
    zf2b = zf2_ref[...]
    zq = zq_ref[...]
    zq2 = zq2_ref[...]
    zqs_ref[...] = zfb + (zq - zfb)
    zq2s_ref[...] = zf2b + (zq2 - zf2b)
    part = jnp.sum((zq - zfb) ** 2) + jnp.sum((zq2 - zf2b) ** 2)
    contrib = part * ((1.0 + BETA) / (nrows * E))
    prev = jnp.where(i == 0, jnp.zeros((1, 1), jnp.float32), loss_ref[...])
    loss_ref[...] = prev + contrib


def _finish(zf, zf2, zq, zq2):
    nrows = zf.shape[0]
    blk = 2048
    row = lambda i: (i, 0)
    full = lambda i: (0, 0)
    return pl.pallas_call(
        functools.partial(_finish_body, nrows),
        grid=(nrows // blk,),
        in_specs=[pl.BlockSpec((blk, E), row)] * 4,
        out_specs=[pl.BlockSpec((blk, E), row),
                   pl.BlockSpec((blk, E), row),
                   pl.BlockSpec((1, 1), full)],
        out_shape=[jax.ShapeDtypeStruct((nrows, E), jnp.float32),
                   jax.ShapeDtypeStruct((nrows, E), jnp.float32),
                   jax.ShapeDtypeStruct((1, 1), jnp.float32)],
    )(zf, zf2, zq, zq2)


def _prep_body(degp_ref, mm1_ref, dinv_ref, hs1_ref):
    deg = jnp.sum(degp_ref[...], axis=0) + 1.0
    dinv = 1.0 / jnp.sqrt(deg)
    dinv_ref[...] = dinv[:, None]
    hs1_ref[...] = mm1_ref[...] * dinv[:, None]


def _prep(degp, mm1):
    return pl.pallas_call(
        _prep_body,
        out_shape=[jax.ShapeDtypeStruct((N, 1), jnp.float32),
                   jax.ShapeDtypeStruct((N, E), jnp.float32)],
    )(degp, mm1)


def _layer2_body(accp_ref, hs1_ref, dinv_ref, b1_ref, w2_ref, hs2_ref):
    dinv = dinv_ref[...]
    h2 = dinv * (accp_ref[0] + accp_ref[1] + hs1_ref[...]) + b1_ref[...]
    h2 = jnp.maximum(h2, 0.0)
    hs2_ref[...] = jnp.dot(h2, w2_ref[...],
                           preferred_element_type=jnp.float32) * dinv


def _layer2(accp1, hs1, dinv, b1_2d, W2):
    return pl.pallas_call(
        _layer2_body,
        out_shape=jax.ShapeDtypeStruct((N, E), jnp.float32),
    )(accp1, hs1, dinv, b1_2d, W2)


def _codebooks_body(accp_ref, hs2_ref, dinv_ref, b2_ref,
                    ew_ref, ew2_ref, ewt_ref, sqall_ref):
    total = (dinv_ref[...] * (accp_ref[0] + accp_ref[1] + hs2_ref[...])
             + b2_ref[...])
    ew = total[:ADJ]
    ew2 = total[ADJ:]
    ew_ref[...] = ew
    ew2_ref[...] = ew2
    zpad = jnp.zeros((E, ADJ), jnp.float32)
    top = jnp.concatenate([ew.T, zpad], axis=1)
    bot = jnp.concatenate([zpad, ew2.T], axis=1)
    ewt_ref[...] = jnp.concatenate([top, bot], axis=0)
    sqall_ref[...] = jnp.concatenate(
        [jnp.sum(ew ** 2, axis=1), jnp.sum(ew2 ** 2, axis=1)])[None, :]


def _codebooks(accp2, hs2, dinv, b2_2d):
    return pl.pallas_call(
        _codebooks_body,
        out_shape=[jax.ShapeDtypeStruct((ADJ, E), jnp.float32),
                   jax.ShapeDtypeStruct((ADJ, E), jnp.float32),
                   jax.ShapeDtypeStruct((2 * E, 2 * ADJ), jnp.float32),
                   jax.ShapeDtypeStruct((1, 2 * ADJ), jnp.float32)],
    )(accp2, hs2, dinv, b2_2d)


def _quant_body(zf_ref, zf2_ref, ewt_ref, sqall_ref,
                i1a_ref, i1b_ref, i2_ref):
    blk = zf_ref.shape[0]
    iota = lax.broadcasted_iota(jnp.int32, (blk, ADJ), 1)
    big = jnp.int32(2 ** 30)
    nn = (((1,), (0,)), ((), ()))

    zfb = zf_ref[...]
    zf2b = zf2_ref[...]
    zcat = jnp.concatenate([zfb, zf2b], axis=1)
    mmall = lax.dot_general(zcat, ewt_ref[...], nn,
                            preferred_element_type=jnp.float32)
    sqall = sqall_ref[...]

    d = (jnp.sum(zfb ** 2, axis=1, keepdims=True) + sqall[:, :ADJ]
         - 2.0 * mmall[:, :ADJ])
    i1 = jnp.argmin(d, axis=1).astype(jnp.int32)
    d2 = jnp.where(iota == i1[:, None], jnp.float32(jnp.inf), d)
    i1b = jnp.argmin(d2, axis=1).astype(jnp.int32)
    i1a_ref[...] = i1[:, None]
    i1b_ref[...] = i1b[:, None]

    dn = (jnp.sum(zf2b ** 2, axis=1, keepdims=True) + sqall[:, ADJ:]
          - 2.0 * mmall[:, ADJ:])
    i2 = jnp.argmin(dn, axis=1).astype(jnp.int32)
    i2_ref[...] = i2[:, None]


def _quant(zf, zf2, ewt, sqall):
    nrows = zf.shape[0]
    blk = 256
    grid = (nrows // blk,)
    full = lambda i: (0, 0)
    row = lambda i: (i, 0)
    return pl.pallas_call(
        _quant_body,
        grid=grid,
        in_specs=[pl.BlockSpec((blk, E), row),
                  pl.BlockSpec((blk, E), row),
                  pl.BlockSpec((2 * E, 2 * ADJ), full),
                  pl.BlockSpec((1, 2 * ADJ), full)],
        out_specs=[pl.BlockSpec((blk, 1), row),
                   pl.BlockSpec((blk, 1), row),
                   pl.BlockSpec((blk, 1), row)],
        out_shape=[jax.ShapeDtypeStruct((nrows, 1), jnp.int32),
                   jax.ShapeDtypeStruct((nrows, 1), jnp.int32),
                   jax.ShapeDtypeStruct((nrows, 1), jnp.int32)],
    )(zf, zf2, ewt, sqall)


def kernel(z, code, edge_index, W1, b1, W2, b2):
    b = z.shape[0]
    src2d = edge_index[0].reshape(-1, EDGE_COLS)
    dst2d = edge_index[1].reshape(-1, EDGE_COLS)

    degp = _sc_degree(dst2d)
    mm1 = _mm1(code, W1)
    dinv, hs1 = _prep(degp, mm1)
    accp1 = _sc_scatter(hs1, src2d, dst2d)
    hs2 = _layer2(accp1, hs1, dinv, b1.reshape(1, E), W2)
    accp2 = _sc_scatter(hs2, src2d, dst2d)
    ew, ew2, ewt, sqall = _codebooks(accp2, hs2, dinv, b2.reshape(1, E))

    zf = jnp.transpose(z[:, :E], (0, 2, 3, 1)).reshape(-1, E)
    zf2 = jnp.transpose(z[:, E:], (0, 2, 3, 1)).reshape(-1, E)
    i1a, i1b, i2 = _quant(zf, zf2, ewt, sqall)
    zqr, zq2r = _sc_gather(ew, ew2,
                           i1a.reshape(-1, EDGE_COLS),
                           i1b.reshape(-1, EDGE_COLS),
                           i2.reshape(-1, EDGE_COLS))
    zq, zq2, lossm = _finish(zf, zf2, zqr, zq2r)

    h, w = z.shape[2], z.shape[3]
    z_adj_q = jnp.transpose(zq.reshape(b, h, w, E), (0, 3, 1, 2))
    z_noun_q = jnp.transpose(zq2.reshape(b, h, w, E), (0, 3, 1, 2))
    z_q = jnp.concatenate([z_adj_q, z_noun_q], axis=1)
    idx1 = jnp.concatenate([i1a, i1b], axis=1).reshape(b, -1)
    idx2 = i2.reshape(b, -1)
    loss = lossm.reshape(())
    return z_q, loss, idx1, idx2


# speedup vs baseline: 1.3094x; 1.0094x over previous
import functools

import jax
import jax.numpy as jnp
from jax import lax
from jax.experimental import pallas as pl
from jax.experimental.pallas import tpu as pltpu
from jax.experimental.pallas import tpu_sc as plsc

E = 32
N = 8192
ADJ = 4096
BETA = 0.25
NC, NS = 2, 16
NW = NC * NS
EDGE_COLS = 128

def _mm1_body(x_ref, w_ref, o_ref):
    o_ref[...] = jnp.dot(x_ref[...], w_ref[...],
                         preferred_element_type=jnp.float32)


def _mm1(code, W1):
    M, K = code.shape
    Nout = W1.shape[1]
    blk = 1024
    return pl.pallas_call(
        _mm1_body,
        grid=(M // blk,),
        in_specs=[pl.BlockSpec((blk, K), lambda i: (i, 0)),
                  pl.BlockSpec((K, Nout), lambda i: (0, 0))],
        out_specs=pl.BlockSpec((blk, Nout), lambda i: (i, 0)),
        out_shape=jax.ShapeDtypeStruct((M, Nout), jnp.float32),
    )(code, W1)


def _sc_degree(dst2d):
    rows_pt = dst2d.shape[0] // NW
    mesh = plsc.VectorSubcoreMesh(core_axis_name="c", subcore_axis_name="s")

    @functools.partial(
        pl.kernel, mesh=mesh,
        out_type=jax.ShapeDtypeStruct((NW, N), jnp.float32),
        scratch_types=[pltpu.VMEM((rows_pt, EDGE_COLS), jnp.int32),
                       pltpu.VMEM((N,), jnp.float32)],
        compiler_params=pltpu.CompilerParams(use_tc_tiling_on_sc=False,
                                             needs_layout_passes=False),
    )
    def k(dst_hbm, out_hbm, dstv, hist):
        c = lax.axis_index("c")
        s = lax.axis_index("s")
        wid = c * NS + s
        z16 = jnp.zeros((16,), jnp.float32)

        def zero_body(i, _):
            hist[pl.ds(i * 16, 16)] = z16
            return 0
        lax.fori_loop(0, N // 16, zero_body, 0)

        pltpu.sync_copy(dst_hbm.at[pl.ds(wid * rows_pt, rows_pt)], dstv)
        ones = jnp.ones((16,), jnp.float32)

        def body(r, _):
            for g in range(EDGE_COLS // 16):
                idx = dstv[r, pl.ds(g * 16, 16)]
                plsc.addupdate_scatter(hist, [idx], ones)
            return 0
        lax.fori_loop(0, rows_pt, body, 0)

        pltpu.sync_copy(hist, out_hbm.at[wid])

    return k(dst2d)


def _sc_scatter(hs, src2d, dst2d):
    rows_pt = src2d.shape[0] // NW
    rows_per_sub = N // NS
    mesh = plsc.VectorSubcoreMesh(core_axis_name="c", subcore_axis_name="s")

    nbuf = 4
    ngrp = rows_pt // nbuf

    @functools.partial(
        pl.kernel, mesh=mesh,
        out_type=jax.ShapeDtypeStruct((NC, N, E), jnp.float32),
        scratch_types=(
            [pltpu.VMEM((rows_pt, EDGE_COLS), jnp.int32),
             pltpu.VMEM((rows_pt, EDGE_COLS), jnp.int32)]
            + [pltpu.VMEM((EDGE_COLS, E), jnp.float32)] * nbuf
            + [pltpu.VMEM((EDGE_COLS, E), jnp.float32),
               pltpu.VMEM_SHARED((N, E), jnp.float32)]
            + [pltpu.SemaphoreType.DMA] * (2 * nbuf)
        ),
        compiler_params=pltpu.CompilerParams(use_tc_tiling_on_sc=False),
    )
    def k(hs_hbm, src_hbm, dst_hbm, out_hbm, srcv, dstv,
          r0, r1, r2, r3, zb, acc,
          g0, g1, g2, g3, s0, s1, s2, s3):
        rows = [r0, r1, r2, r3]
        gs = [g0, g1, g2, g3]
        ss = [s0, s1, s2, s3]
        c = lax.axis_index("c")
        s = lax.axis_index("s")
        wid = c * NS + s
        z16 = jnp.zeros((16,), jnp.float32)

        def zb_body(i, _):
            zb[i, pl.ds(0, 16)] = z16
            zb[i, pl.ds(16, 16)] = z16
            return 0
        lax.fori_loop(0, EDGE_COLS, zb_body, 0)
        for t in range(rows_per_sub // EDGE_COLS):
            pltpu.sync_copy(zb, acc.at[pl.ds(s * rows_per_sub + t * EDGE_COLS,
                                             EDGE_COLS)])
        pltpu.sync_copy(src_hbm.at[pl.ds(wid * rows_pt, rows_pt)], srcv)
        pltpu.sync_copy(dst_hbm.at[pl.ds(wid * rows_pt, rows_pt)], dstv)
        plsc.subcore_barrier()

        for b in range(nbuf):
            pltpu.async_copy(hs_hbm.at[srcv.at[b]], rows[b], gs[b])

        def grp(g, _):
            j = g * nbuf
            for b in range(nbuf):
                pltpu.make_async_copy(hs_hbm.at[srcv.at[j + b]],
                                      rows[b], gs[b]).wait()
                pltpu.async_copy(rows[b], acc.at[dstv.at[j + b]], ss[b],
                                 add=True)
            for b in range(nbuf):
                @pl.when(g < ngrp - 1)
                def _():
                    pltpu.make_async_copy(rows[b], acc.at[dstv.at[j + b]],
                                          ss[b]).wait()
                    pltpu.async_copy(hs_hbm.at[srcv.at[j + nbuf + b]],
                                     rows[b], gs[b])
            return 0
        lax.fori_loop(0, ngrp, grp, 0)
        for b in range(nbuf):
            pltpu.make_async_copy(rows[b],
                                  acc.at[dstv.at[(ngrp - 1) * nbuf + b]],
                                  ss[b]).wait()
        plsc.subcore_barrier()

        pltpu.sync_copy(acc.at[pl.ds(s * rows_per_sub, rows_per_sub)],
                        out_hbm.at[c, pl.ds(s * rows_per_sub, rows_per_sub)])

    return k(hs, src2d, dst2d)


def _sc_gather(ew, ew2, i1a2d, i1b2d, i22d):
    nq = i1a2d.shape[0] * i1a2d.shape[1]
    rows_pt = i1a2d.shape[0] // NW
    mesh = plsc.VectorSubcoreMesh(core_axis_name="c", subcore_axis_name="s")

    @functools.partial(
        pl.kernel, mesh=mesh,
        out_type=[jax.ShapeDtypeStruct((nq, E), jnp.float32),
                  jax.ShapeDtypeStruct((nq, E), jnp.float32)],
        scratch_types=[
            pltpu.VMEM((rows_pt, EDGE_COLS), jnp.int32),
            pltpu.VMEM((rows_pt, EDGE_COLS), jnp.int32),
            pltpu.VMEM((rows_pt, EDGE_COLS), jnp.int32),
            pltpu.VMEM((EDGE_COLS, E), jnp.float32),
            pltpu.VMEM((EDGE_COLS, E), jnp.float32),
            pltpu.VMEM((EDGE_COLS, E), jnp.float32),
            pltpu.SemaphoreType.DMA,
        ],
        compiler_params=pltpu.CompilerParams(use_tc_tiling_on_sc=False),
    )
    def k(ew_hbm, ew2_hbm, ia_hbm, ib_hbm, i2_hbm, zq_hbm, zq2_hbm,
          iav, ibv, i2v, r1, r2, r3, sem):
        c = lax.axis_index("c")
        s = lax.axis_index("s")
        wid = c * NS + s
        pltpu.sync_copy(ia_hbm.at[pl.ds(wid * rows_pt, rows_pt)], iav)
        pltpu.sync_copy(ib_hbm.at[pl.ds(wid * rows_pt, rows_pt)], ibv)
        pltpu.sync_copy(i2_hbm.at[pl.ds(wid * rows_pt, rows_pt)], i2v)

        def body(j, _):
            base = (wid * rows_pt + j) * EDGE_COLS
            pltpu.async_copy(ew_hbm.at[iav.at[j]], r1, sem).wait()
            pltpu.async_copy(ew_hbm.at[ibv.at[j]], r2, sem).wait()
            pltpu.async_copy(ew2_hbm.at[i2v.at[j]], r3, sem).wait()

            def row_body(r, _):
                for cc in range(E // 16):
                    sl = pl.ds(cc * 16, 16)
                    r1[r, sl] = (r1[r, sl] + r2[r, sl]) * 0.5
                return 0
            lax.fori_loop(0, EDGE_COLS, row_body, 0)
            pltpu.sync_copy(r1, zq_hbm.at[pl.ds(base, EDGE_COLS)])
            pltpu.sync_copy(r3, zq2_hbm.at[pl.ds(base, EDGE_COLS)])
            return 0
        lax.fori_loop(0, rows_pt, body, 0)

    return k(ew, ew2, i1a2d, i1b2d, i22d)


def _finish_body(nrows, zf_ref, zf2_ref, zq_ref, zq2_ref,
                 zqs_ref, zq2s_ref, loss_ref):
    i = pl.program_id(0)
    zfb = zf_ref[...]
    zf2b = zf2_ref[...]
    zq = zq_ref[...]
    zq2 = zq2_ref[...]
    zqs_ref[...] = zfb + (zq - zfb)
    zq2s_ref[...] = zf2b + (zq2 - zf2b)
    part = jnp.sum((zq - zfb) ** 2) + jnp.sum((zq2 - zf2b) ** 2)
    contrib = part * ((1.0 + BETA) / (nrows * E))
    prev = jnp.where(i == 0, jnp.zeros((1, 1), jnp.float32), loss_ref[...])
    loss_ref[...] = prev + contrib


def _finish(zf, zf2, zq, zq2):
    nrows = zf.shape[0]
    blk = 2048
    row = lambda i: (i, 0)
    full = lambda i: (0, 0)
    return pl.pallas_call(
        functools.partial(_finish_body, nrows),
        grid=(nrows // blk,),
        in_specs=[pl.BlockSpec((blk, E), row)] * 4,
        out_specs=[pl.BlockSpec((blk, E), row),
                   pl.BlockSpec((blk, E), row),
                   pl.BlockSpec((1, 1), full)],
        out_shape=[jax.ShapeDtypeStruct((nrows, E), jnp.float32),
                   jax.ShapeDtypeStruct((nrows, E), jnp.float32),
                   jax.ShapeDtypeStruct((1, 1), jnp.float32)],
    )(zf, zf2, zq, zq2)


def _prep_body(degp_ref, mm1_ref, dinv_ref, hs1_ref):
    deg = jnp.sum(degp_ref[...], axis=0) + 1.0
    dinv = 1.0 / jnp.sqrt(deg)
    dinv_ref[...] = dinv[:, None]
    hs1_ref[...] = mm1_ref[...] * dinv[:, None]


def _prep(degp, mm1):
    return pl.pallas_call(
        _prep_body,
        out_shape=[jax.ShapeDtypeStruct((N, 1), jnp.float32),
                   jax.ShapeDtypeStruct((N, E), jnp.float32)],
    )(degp, mm1)


def _layer2_body(accp_ref, hs1_ref, dinv_ref, b1_ref, w2_ref, hs2_ref):
    dinv = dinv_ref[...]
    h2 = dinv * (accp_ref[0] + accp_ref[1] + hs1_ref[...]) + b1_ref[...]
    h2 = jnp.maximum(h2, 0.0)
    hs2_ref[...] = jnp.dot(h2, w2_ref[...],
                           preferred_element_type=jnp.float32) * dinv


def _layer2(accp1, hs1, dinv, b1_2d, W2):
    return pl.pallas_call(
        _layer2_body,
        out_shape=jax.ShapeDtypeStruct((N, E), jnp.float32),
    )(accp1, hs1, dinv, b1_2d, W2)


def _codebooks_body(accp_ref, hs2_ref, dinv_ref, b2_ref,
                    ew_ref, ew2_ref, ewt_ref, sqall_ref):
    total = (dinv_ref[...] * (accp_ref[0] + accp_ref[1] + hs2_ref[...])
             + b2_ref[...])
    ew = total[:ADJ]
    ew2 = total[ADJ:]
    ew_ref[...] = ew
    ew2_ref[...] = ew2
    zpad = jnp.zeros((E, ADJ), jnp.float32)
    top = jnp.concatenate([ew.T, zpad], axis=1)
    bot = jnp.concatenate([zpad, ew2.T], axis=1)
    ewt_ref[...] = jnp.concatenate([top, bot], axis=0)
    sqall_ref[...] = jnp.concatenate(
        [jnp.sum(ew ** 2, axis=1), jnp.sum(ew2 ** 2, axis=1)])[None, :]


def _codebooks(accp2, hs2, dinv, b2_2d):
    return pl.pallas_call(
        _codebooks_body,
        out_shape=[jax.ShapeDtypeStruct((ADJ, E), jnp.float32),
                   jax.ShapeDtypeStruct((ADJ, E), jnp.float32),
                   jax.ShapeDtypeStruct((2 * E, 2 * ADJ), jnp.float32),
                   jax.ShapeDtypeStruct((1, 2 * ADJ), jnp.float32)],
    )(accp2, hs2, dinv, b2_2d)


def _quant_body(zf_ref, zf2_ref, ewt_ref, sqall_ref,
                i1a_ref, i1b_ref, i2_ref):
    blk = zf_ref.shape[0]
    iota = lax.broadcasted_iota(jnp.int32, (blk, ADJ), 1)
    big = jnp.int32(2 ** 30)
    nn = (((1,), (0,)), ((), ()))

    zfb = zf_ref[...]
    zf2b = zf2_ref[...]
    zcat = jnp.concatenate([zfb, zf2b], axis=1)
    mmall = lax.dot_general(zcat, ewt_ref[...], nn,
                            preferred_element_type=jnp.float32)
    sqall = sqall_ref[...]

    d = (jnp.sum(zfb ** 2, axis=1, keepdims=True) + sqall[:, :ADJ]
         - 2.0 * mmall[:, :ADJ])
    m1 = jnp.min(d, axis=1, keepdims=True)
    i1 = jnp.min(jnp.where(d == m1, iota, big), axis=1)
    d2 = jnp.where(iota == i1[:, None], jnp.float32(jnp.inf), d)
    m2 = jnp.min(d2, axis=1, keepdims=True)
    i1b = jnp.min(jnp.where(d2 == m2, iota, big), axis=1)
    i1a_ref[...] = i1[:, None]
    i1b_ref[...] = i1b[:, None]

    dn = (jnp.sum(zf2b ** 2, axis=1, keepdims=True) + sqall[:, ADJ:]
          - 2.0 * mmall[:, ADJ:])
    mn = jnp.min(dn, axis=1, keepdims=True)
    i2 = jnp.min(jnp.where(dn == mn, iota, big), axis=1)
    i2_ref[...] = i2[:, None]


def _quant(zf, zf2, ewt, sqall):
    nrows = zf.shape[0]
    blk = 256
    grid = (nrows // blk,)
    full = lambda i: (0, 0)
    row = lambda i: (i, 0)
    return pl.pallas_call(
        _quant_body,
        grid=grid,
        in_specs=[pl.BlockSpec((blk, E), row),
                  pl.BlockSpec((blk, E), row),
                  pl.BlockSpec((2 * E, 2 * ADJ), full),
                  pl.BlockSpec((1, 2 * ADJ), full)],
        out_specs=[pl.BlockSpec((blk, 1), row),
                   pl.BlockSpec((blk, 1), row),
                   pl.BlockSpec((blk, 1), row)],
        out_shape=[jax.ShapeDtypeStruct((nrows, 1), jnp.int32),
                   jax.ShapeDtypeStruct((nrows, 1), jnp.int32),
                   jax.ShapeDtypeStruct((nrows, 1), jnp.int32)],
    )(zf, zf2, ewt, sqall)


def kernel(z, code, edge_index, W1, b1, W2, b2):
    b = z.shape[0]
    src2d = edge_index[0].reshape(-1, EDGE_COLS)
    dst2d = edge_index[1].reshape(-1, EDGE_COLS)

    degp = _sc_degree(dst2d)
    mm1 = _mm1(code, W1)
    dinv, hs1 = _prep(degp, mm1)
    accp1 = _sc_scatter(hs1, src2d, dst2d)
    hs2 = _layer2(accp1, hs1, dinv, b1.reshape(1, E), W2)
    accp2 = _sc_scatter(hs2, src2d, dst2d)
    ew, ew2, ewt, sqall = _codebooks(accp2, hs2, dinv, b2.reshape(1, E))

    zf = jnp.transpose(z[:, :E], (0, 2, 3, 1)).reshape(-1, E)
    zf2 = jnp.transpose(z[:, E:], (0, 2, 3, 1)).reshape(-1, E)
    i1a, i1b, i2 = _quant(zf, zf2, ewt, sqall)
    zqr, zq2r = _sc_gather(ew, ew2,
                           i1a.reshape(-1, EDGE_COLS),
                           i1b.reshape(-1, EDGE_COLS),
                           i2.reshape(-1, EDGE_COLS))
    zq, zq2, lossm = _finish(zf, zf2, zqr, zq2r)

    h, w = z.shape[2], z.shape[3]
    z_adj_q = jnp.transpose(zq.reshape(b, h, w, E), (0, 3, 1, 2))
    z_noun_q = jnp.transpose(zq2.reshape(b, h, w, E), (0, 3, 1, 2))
    z_q = jnp.concatenate([z_adj_q, z_noun_q], axis=1)
    idx1 = jnp.concatenate([i1a, i1b], axis=1).reshape(b, -1)
    idx2 = i2.reshape(b, -1)
    loss = lossm.reshape(())
    return z_q, loss, idx1, idx2


# speedup vs baseline: 1.4380x; 1.0982x over previous
import functools

import jax
import jax.numpy as jnp
from jax import lax
from jax.experimental import pallas as pl
from jax.experimental.pallas import tpu as pltpu
from jax.experimental.pallas import tpu_sc as plsc

E = 32
N = 8192
ADJ = 4096
BETA = 0.25
NC, NS = 2, 16
NW = NC * NS
EDGE_COLS = 128

def _mm1_body(x_ref, w_ref, o_ref):
    o_ref[...] = jnp.dot(x_ref[...], w_ref[...],
                         preferred_element_type=jnp.float32)


def _mm1(code, W1):
    M, K = code.shape
    Nout = W1.shape[1]
    blk = 1024
    return pl.pallas_call(
        _mm1_body,
        grid=(M // blk,),
        in_specs=[pl.BlockSpec((blk, K), lambda i: (i, 0)),
                  pl.BlockSpec((K, Nout), lambda i: (0, 0))],
        out_specs=pl.BlockSpec((blk, Nout), lambda i: (i, 0)),
        out_shape=jax.ShapeDtypeStruct((M, Nout), jnp.float32),
    )(code, W1)


def _sc_degree(dst2d):
    rows_pt = dst2d.shape[0] // NW
    mesh = plsc.VectorSubcoreMesh(core_axis_name="c", subcore_axis_name="s")

    @functools.partial(
        pl.kernel, mesh=mesh,
        out_type=jax.ShapeDtypeStruct((NW, N), jnp.float32),
        scratch_types=[pltpu.VMEM((rows_pt, EDGE_COLS), jnp.int32),
                       pltpu.VMEM((N,), jnp.float32)],
        compiler_params=pltpu.CompilerParams(use_tc_tiling_on_sc=False,
                                             needs_layout_passes=False),
    )
    def k(dst_hbm, out_hbm, dstv, hist):
        c = lax.axis_index("c")
        s = lax.axis_index("s")
        wid = c * NS + s
        z16 = jnp.zeros((16,), jnp.float32)

        def zero_body(i, _):
            hist[pl.ds(i * 16, 16)] = z16
            return 0
        lax.fori_loop(0, N // 16, zero_body, 0)

        pltpu.sync_copy(dst_hbm.at[pl.ds(wid * rows_pt, rows_pt)], dstv)
        ones = jnp.ones((16,), jnp.float32)

        def body(r, _):
            for g in range(EDGE_COLS // 16):
                idx = dstv[r, pl.ds(g * 16, 16)]
                plsc.addupdate_scatter(hist, [idx], ones)
            return 0
        lax.fori_loop(0, rows_pt, body, 0)

        pltpu.sync_copy(hist, out_hbm.at[wid])

    return k(dst2d)


def _sc_scatter(hs, src2d, dst2d):
    rows_pt = src2d.shape[0] // NW
    rows_per_sub = N // NS
    mesh = plsc.VectorSubcoreMesh(core_axis_name="c", subcore_axis_name="s")

    nbuf = 4
    ngrp = rows_pt // nbuf

    @functools.partial(
        pl.kernel, mesh=mesh,
        out_type=jax.ShapeDtypeStruct((NC, N, E), jnp.float32),
        scratch_types=(
            [pltpu.VMEM((rows_pt, EDGE_COLS), jnp.int32),
             pltpu.VMEM((rows_pt, EDGE_COLS), jnp.int32)]
            + [pltpu.VMEM((EDGE_COLS, E), jnp.float32)] * nbuf
            + [pltpu.VMEM((EDGE_COLS, E), jnp.float32),
               pltpu.VMEM_SHARED((N, E), jnp.float32)]
            + [pltpu.SemaphoreType.DMA] * (2 * nbuf)
        ),
        compiler_params=pltpu.CompilerParams(use_tc_tiling_on_sc=False),
    )
    def k(hs_hbm, src_hbm, dst_hbm, out_hbm, srcv, dstv,
          r0, r1, r2, r3, zb, acc,
          g0, g1, g2, g3, s0, s1, s2, s3):
        rows = [r0, r1, r2, r3]
        gs = [g0, g1, g2, g3]
        ss = [s0, s1, s2, s3]
        c = lax.axis_index("c")
        s = lax.axis_index("s")
        wid = c * NS + s
        z16 = jnp.zeros((16,), jnp.float32)

        def zb_body(i, _):
            zb[i, pl.ds(0, 16)] = z16
            zb[i, pl.ds(16, 16)] = z16
            return 0
        lax.fori_loop(0, EDGE_COLS, zb_body, 0)
        for t in range(rows_per_sub // EDGE_COLS):
            pltpu.sync_copy(zb, acc.at[pl.ds(s * rows_per_sub + t * EDGE_COLS,
                                             EDGE_COLS)])
        pltpu.sync_copy(src_hbm.at[pl.ds(wid * rows_pt, rows_pt)], srcv)
        pltpu.sync_copy(dst_hbm.at[pl.ds(wid * rows_pt, rows_pt)], dstv)
        plsc.subcore_barrier()

        for b in range(nbuf):
            pltpu.async_copy(hs_hbm.at[srcv.at[b]], rows[b], gs[b])

        def grp(g, _):
            j = g * nbuf
            for b in range(nbuf):
                pltpu.make_async_copy(hs_hbm.at[srcv.at[j + b]],
                                      rows[b], gs[b]).wait()
                pltpu.async_copy(rows[b], acc.at[dstv.at[j + b]], ss[b],
                                 add=True)
            for b in range(nbuf):
                @pl.when(g < ngrp - 1)
                def _():
                    pltpu.make_async_copy(rows[b], acc.at[dstv.at[j + b]],
                                          ss[b]).wait()
                    pltpu.async_copy(hs_hbm.at[srcv.at[j + nbuf + b]],
                                     rows[b], gs[b])
            return 0
        lax.fori_loop(0, ngrp, grp, 0)
        for b in range(nbuf):
            pltpu.make_async_copy(rows[b],
                                  acc.at[dstv.at[(ngrp - 1) * nbuf + b]],
                                  ss[b]).wait()
        plsc.subcore_barrier()

        pltpu.sync_copy(acc.at[pl.ds(s * rows_per_sub, rows_per_sub)],
                        out_hbm.at[c, pl.ds(s * rows_per_sub, rows_per_sub)])

    return k(hs, src2d, dst2d)


def _sc_gather(ew, ew2, i1a2d, i1b2d, i22d):
    nq = i1a2d.shape[0] * i1a2d.shape[1]
    rows_pt = i1a2d.shape[0] // NW
    mesh = plsc.VectorSubcoreMesh(core_axis_name="c", subcore_axis_name="s")

    @functools.partial(
        pl.kernel, mesh=mesh,
        out_type=[jax.ShapeDtypeStruct((nq, E), jnp.float32),
                  jax.ShapeDtypeStruct((nq, E), jnp.float32)],
        scratch_types=[
            pltpu.VMEM((rows_pt, EDGE_COLS), jnp.int32),
            pltpu.VMEM((rows_pt, EDGE_COLS), jnp.int32),
            pltpu.VMEM((rows_pt, EDGE_COLS), jnp.int32),
            pltpu.VMEM((EDGE_COLS, E), jnp.float32),
            pltpu.VMEM((EDGE_COLS, E), jnp.float32),
            pltpu.VMEM((EDGE_COLS, E), jnp.float32),
            pltpu.SemaphoreType.DMA,
        ],
        compiler_params=pltpu.CompilerParams(use_tc_tiling_on_sc=False),
    )
    def k(ew_hbm, ew2_hbm, ia_hbm, ib_hbm, i2_hbm, zq_hbm, zq2_hbm,
          iav, ibv, i2v, r1, r2, r3, sem):
        c = lax.axis_index("c")
        s = lax.axis_index("s")
        wid = c * NS + s
        pltpu.sync_copy(ia_hbm.at[pl.ds(wid * rows_pt, rows_pt)], iav)
        pltpu.sync_copy(ib_hbm.at[pl.ds(wid * rows_pt, rows_pt)], ibv)
        pltpu.sync_copy(i2_hbm.at[pl.ds(wid * rows_pt, rows_pt)], i2v)

        def body(j, _):
            base = (wid * rows_pt + j) * EDGE_COLS
            pltpu.async_copy(ew_hbm.at[iav.at[j]], r1, sem).wait()
            pltpu.async_copy(ew_hbm.at[ibv.at[j]], r2, sem).wait()
            pltpu.async_copy(ew2_hbm.at[i2v.at[j]], r3, sem).wait()

            def row_body(r, _):
                for cc in range(E // 16):
                    sl = pl.ds(cc * 16, 16)
                    r1[r, sl] = (r1[r, sl] + r2[r, sl]) * 0.5
                return 0
            lax.fori_loop(0, EDGE_COLS, row_body, 0)
            pltpu.sync_copy(r1, zq_hbm.at[pl.ds(base, EDGE_COLS)])
            pltpu.sync_copy(r3, zq2_hbm.at[pl.ds(base, EDGE_COLS)])
            return 0
        lax.fori_loop(0, rows_pt, body, 0)

    return k(ew, ew2, i1a2d, i1b2d, i22d)


def _finish_body(nrows, zfall_ref, zq_ref, zq2_ref,
                 zqs_ref, zq2s_ref, loss_ref):
    i = pl.program_id(0)
    zfb = zfall_ref[:, :E]
    zf2b = zfall_ref[:, E:]
    zq = zq_ref[...]
    zq2 = zq2_ref[...]
    zqs_ref[...] = zfb + (zq - zfb)
    zq2s_ref[...] = zf2b + (zq2 - zf2b)
    part = jnp.sum((zq - zfb) ** 2) + jnp.sum((zq2 - zf2b) ** 2)
    contrib = part * ((1.0 + BETA) / (nrows * E))
    prev = jnp.where(i == 0, jnp.zeros((1, 1), jnp.float32), loss_ref[...])
    loss_ref[...] = prev + contrib


def _finish(zfall, zq, zq2):
    nrows = zfall.shape[0]
    blk = 2048
    row = lambda i: (i, 0)
    full = lambda i: (0, 0)
    return pl.pallas_call(
        functools.partial(_finish_body, nrows),
        grid=(nrows // blk,),
        in_specs=[pl.BlockSpec((blk, 2 * E), row),
                  pl.BlockSpec((blk, E), row),
                  pl.BlockSpec((blk, E), row)],
        out_specs=[pl.BlockSpec((blk, E), row),
                   pl.BlockSpec((blk, E), row),
                   pl.BlockSpec((1, 1), full)],
        out_shape=[jax.ShapeDtypeStruct((nrows, E), jnp.float32),
                   jax.ShapeDtypeStruct((nrows, E), jnp.float32),
                   jax.ShapeDtypeStruct((1, 1), jnp.float32)],
    )(zfall, zq, zq2)


def _prep_body(degp_ref, mm1_ref, dinv_ref, hs1_ref):
    deg = jnp.sum(degp_ref[...], axis=0) + 1.0
    dinv = 1.0 / jnp.sqrt(deg)
    dinv_ref[...] = dinv[:, None]
    hs1_ref[...] = mm1_ref[...] * dinv[:, None]


def _prep(degp, mm1):
    return pl.pallas_call(
        _prep_body,
        out_shape=[jax.ShapeDtypeStruct((N, 1), jnp.float32),
                   jax.ShapeDtypeStruct((N, E), jnp.float32)],
    )(degp, mm1)


def _layer2_body(accp_ref, hs1_ref, dinv_ref, b1_ref, w2_ref, hs2_ref):
    dinv = dinv_ref[...]
    h2 = dinv * (accp_ref[0] + accp_ref[1] + hs1_ref[...]) + b1_ref[...]
    h2 = jnp.maximum(h2, 0.0)
    hs2_ref[...] = jnp.dot(h2, w2_ref[...],
                           preferred_element_type=jnp.float32) * dinv


def _layer2(accp1, hs1, dinv, b1_2d, W2):
    return pl.pallas_call(
        _layer2_body,
        out_shape=jax.ShapeDtypeStruct((N, E), jnp.float32),
    )(accp1, hs1, dinv, b1_2d, W2)


def _codebooks_body(accp_ref, hs2_ref, dinv_ref, b2_ref,
                    ew_ref, ew2_ref, ewt_ref, sqall_ref):
    total = (dinv_ref[...] * (accp_ref[0] + accp_ref[1] + hs2_ref[...])
             + b2_ref[...])
    ew = total[:ADJ]
    ew2 = total[ADJ:]
    ew_ref[...] = ew
    ew2_ref[...] = ew2
    zpad = jnp.zeros((E, ADJ), jnp.float32)
    top = jnp.concatenate([ew.T, zpad], axis=1)
    bot = jnp.concatenate([zpad, ew2.T], axis=1)
    ewt_ref[...] = jnp.concatenate([top, bot], axis=0)
    sqall_ref[...] = jnp.concatenate(
        [jnp.sum(ew ** 2, axis=1), jnp.sum(ew2 ** 2, axis=1)])[None, :]


def _codebooks(accp2, hs2, dinv, b2_2d):
    return pl.pallas_call(
        _codebooks_body,
        out_shape=[jax.ShapeDtypeStruct((ADJ, E), jnp.float32),
                   jax.ShapeDtypeStruct((ADJ, E), jnp.float32),
                   jax.ShapeDtypeStruct((2 * E, 2 * ADJ), jnp.float32),
                   jax.ShapeDtypeStruct((1, 2 * ADJ), jnp.float32)],
    )(accp2, hs2, dinv, b2_2d)


def _quant_body(zfall_ref, ewt_ref, sqall_ref, i1a_ref, i1b_ref, i2_ref):
    blk = zfall_ref.shape[0]
    fiota = lax.broadcasted_iota(jnp.int32, (1, ADJ), 1).astype(jnp.float32)
    big = jnp.float32(1e9)
    nn = (((1,), (0,)), ((), ()))

    zfall = zfall_ref[...]
    zfb = zfall[:, :E]
    zf2b = zfall[:, E:]
    mmall = lax.dot_general(zfall, ewt_ref[...], nn,
                            preferred_element_type=jnp.float32)
    sqall = sqall_ref[...]

    d = (jnp.sum(zfb ** 2, axis=1, keepdims=True) + sqall[:, :ADJ]
         - 2.0 * mmall[:, :ADJ])
    m1 = jnp.min(d, axis=1, keepdims=True)
    i1f = jnp.min(jnp.where(d == m1, fiota, big), axis=1)
    d2 = jnp.where(fiota == i1f[:, None], jnp.float32(jnp.inf), d)
    m2 = jnp.min(d2, axis=1, keepdims=True)
    i1bf = jnp.min(jnp.where(d2 == m2, fiota, big), axis=1)
    i1a_ref[...] = i1f[:, None].astype(jnp.int32)
    i1b_ref[...] = i1bf[:, None].astype(jnp.int32)

    dn = (jnp.sum(zf2b ** 2, axis=1, keepdims=True) + sqall[:, ADJ:]
          - 2.0 * mmall[:, ADJ:])
    mn = jnp.min(dn, axis=1, keepdims=True)
    i2f = jnp.min(jnp.where(dn == mn, fiota, big), axis=1)
    i2_ref[...] = i2f[:, None].astype(jnp.int32)


def _quant(zfall, ewt, sqall):
    nrows = zfall.shape[0]
    blk = 256
    grid = (nrows // blk,)
    full = lambda i: (0, 0)
    row = lambda i: (i, 0)
    return pl.pallas_call(
        _quant_body,
        grid=grid,
        in_specs=[pl.BlockSpec((blk, 2 * E), row),
                  pl.BlockSpec((2 * E, 2 * ADJ), full),
                  pl.BlockSpec((1, 2 * ADJ), full)],
        out_specs=[pl.BlockSpec((blk, 1), row),
                   pl.BlockSpec((blk, 1), row),
                   pl.BlockSpec((blk, 1), row)],
        out_shape=[jax.ShapeDtypeStruct((nrows, 1), jnp.int32),
                   jax.ShapeDtypeStruct((nrows, 1), jnp.int32),
                   jax.ShapeDtypeStruct((nrows, 1), jnp.int32)],
    )(zfall, ewt, sqall)


def kernel(z, code, edge_index, W1, b1, W2, b2):
    b = z.shape[0]
    src2d = edge_index[0].reshape(-1, EDGE_COLS)
    dst2d = edge_index[1].reshape(-1, EDGE_COLS)

    degp = _sc_degree(dst2d)
    mm1 = _mm1(code, W1)
    dinv, hs1 = _prep(degp, mm1)
    accp1 = _sc_scatter(hs1, src2d, dst2d)
    hs2 = _layer2(accp1, hs1, dinv, b1.reshape(1, E), W2)
    accp2 = _sc_scatter(hs2, src2d, dst2d)
    ew, ew2, ewt, sqall = _codebooks(accp2, hs2, dinv, b2.reshape(1, E))

    zfall = jnp.transpose(z, (0, 2, 3, 1)).reshape(-1, 2 * E)
    i1a, i1b, i2 = _quant(zfall, ewt, sqall)
    zqr, zq2r = _sc_gather(ew, ew2,
                           i1a.reshape(-1, EDGE_COLS),
                           i1b.reshape(-1, EDGE_COLS),
                           i2.reshape(-1, EDGE_COLS))
    zq, zq2, lossm = _finish(zfall, zqr, zq2r)

    h, w = z.shape[2], z.shape[3]
    z_adj_q = jnp.transpose(zq.reshape(b, h, w, E), (0, 3, 1, 2))
    z_noun_q = jnp.transpose(zq2.reshape(b, h, w, E), (0, 3, 1, 2))
    z_q = jnp.concatenate([z_adj_q, z_noun_q], axis=1)
    idx1 = jnp.concatenate([i1a, i1b], axis=1).reshape(b, -1)
    idx2 = i2.reshape(b, -1)
    loss = lossm.reshape(())
    return z_q, loss, idx1, idx2


# speedup vs baseline: 1.5048x; 1.0465x over previous
import functools

import jax
import jax.numpy as jnp
from jax import lax
from jax.experimental import pallas as pl
from jax.experimental.pallas import tpu as pltpu
from jax.experimental.pallas import tpu_sc as plsc

E = 32
N = 8192
ADJ = 4096
BETA = 0.25
NC, NS = 2, 16
NW = NC * NS
EDGE_COLS = 128

def _mm1_body(x_ref, w_ref, o_ref):
    o_ref[...] = jnp.dot(x_ref[...], w_ref[...],
                         preferred_element_type=jnp.float32)


def _mm1(code, W1):
    M, K = code.shape
    Nout = W1.shape[1]
    blk = 1024
    return pl.pallas_call(
        _mm1_body,
        grid=(M // blk,),
        in_specs=[pl.BlockSpec((blk, K), lambda i: (i, 0)),
                  pl.BlockSpec((K, Nout), lambda i: (0, 0))],
        out_specs=pl.BlockSpec((blk, Nout), lambda i: (i, 0)),
        out_shape=jax.ShapeDtypeStruct((M, Nout), jnp.float32),
    )(code, W1)


def _sc_degree(dst2d):
    rows_pt = dst2d.shape[0] // NW
    mesh = plsc.VectorSubcoreMesh(core_axis_name="c", subcore_axis_name="s")

    @functools.partial(
        pl.kernel, mesh=mesh,
        out_type=jax.ShapeDtypeStruct((NW, N), jnp.float32),
        scratch_types=[pltpu.VMEM((rows_pt, EDGE_COLS), jnp.int32),
                       pltpu.VMEM((N,), jnp.float32)],
        compiler_params=pltpu.CompilerParams(use_tc_tiling_on_sc=False,
                                             needs_layout_passes=False),
    )
    def k(dst_hbm, out_hbm, dstv, hist):
        c = lax.axis_index("c")
        s = lax.axis_index("s")
        wid = c * NS + s
        z16 = jnp.zeros((16,), jnp.float32)

        def zero_body(i, _):
            hist[pl.ds(i * 16, 16)] = z16
            return 0
        lax.fori_loop(0, N // 16, zero_body, 0)

        pltpu.sync_copy(dst_hbm.at[pl.ds(wid * rows_pt, rows_pt)], dstv)
        ones = jnp.ones((16,), jnp.float32)

        def body(r, _):
            for g in range(EDGE_COLS // 16):
                idx = dstv[r, pl.ds(g * 16, 16)]
                plsc.addupdate_scatter(hist, [idx], ones)
            return 0
        lax.fori_loop(0, rows_pt, body, 0)

        pltpu.sync_copy(hist, out_hbm.at[wid])

    return k(dst2d)


def _sc_scatter(hs, src2d, dst2d):
    rows_pt = src2d.shape[0] // NW
    rows_per_sub = N // NS
    mesh = plsc.VectorSubcoreMesh(core_axis_name="c", subcore_axis_name="s")

    nbuf = 4
    ngrp = rows_pt // nbuf

    @functools.partial(
        pl.kernel, mesh=mesh,
        out_type=jax.ShapeDtypeStruct((NC, N, E), jnp.float32),
        scratch_types=(
            [pltpu.VMEM((rows_pt, EDGE_COLS), jnp.int32),
             pltpu.VMEM((rows_pt, EDGE_COLS), jnp.int32)]
            + [pltpu.VMEM((EDGE_COLS, E), jnp.float32)] * nbuf
            + [pltpu.VMEM((EDGE_COLS, E), jnp.float32),
               pltpu.VMEM_SHARED((N, E), jnp.float32)]
            + [pltpu.SemaphoreType.DMA] * (2 * nbuf)
        ),
        compiler_params=pltpu.CompilerParams(use_tc_tiling_on_sc=False),
    )
    def k(hs_hbm, src_hbm, dst_hbm, out_hbm, srcv, dstv,
          r0, r1, r2, r3, zb, acc,
          g0, g1, g2, g3, s0, s1, s2, s3):
        rows = [r0, r1, r2, r3]
        gs = [g0, g1, g2, g3]
        ss = [s0, s1, s2, s3]
        c = lax.axis_index("c")
        s = lax.axis_index("s")
        wid = c * NS + s
        z16 = jnp.zeros((16,), jnp.float32)

        def zb_body(i, _):
            zb[i, pl.ds(0, 16)] = z16
            zb[i, pl.ds(16, 16)] = z16
            return 0
        lax.fori_loop(0, EDGE_COLS, zb_body, 0)
        for t in range(rows_per_sub // EDGE_COLS):
            pltpu.sync_copy(zb, acc.at[pl.ds(s * rows_per_sub + t * EDGE_COLS,
                                             EDGE_COLS)])
        pltpu.sync_copy(src_hbm.at[pl.ds(wid * rows_pt, rows_pt)], srcv)
        pltpu.sync_copy(dst_hbm.at[pl.ds(wid * rows_pt, rows_pt)], dstv)
        plsc.subcore_barrier()

        for b in range(nbuf):
            pltpu.async_copy(hs_hbm.at[srcv.at[b]], rows[b], gs[b])

        def grp(g, _):
            j = g * nbuf
            for b in range(nbuf):
                pltpu.make_async_copy(hs_hbm.at[srcv.at[j + b]],
                                      rows[b], gs[b]).wait()
                pltpu.async_copy(rows[b], acc.at[dstv.at[j + b]], ss[b],
                                 add=True)
            for b in range(nbuf):
                @pl.when(g < ngrp - 1)
                def _():
                    pltpu.make_async_copy(rows[b], acc.at[dstv.at[j + b]],
                                          ss[b]).wait()
                    pltpu.async_copy(hs_hbm.at[srcv.at[j + nbuf + b]],
                                     rows[b], gs[b])
            return 0
        lax.fori_loop(0, ngrp, grp, 0)
        for b in range(nbuf):
            pltpu.make_async_copy(rows[b],
                                  acc.at[dstv.at[(ngrp - 1) * nbuf + b]],
                                  ss[b]).wait()
        plsc.subcore_barrier()

        pltpu.sync_copy(acc.at[pl.ds(s * rows_per_sub, rows_per_sub)],
                        out_hbm.at[c, pl.ds(s * rows_per_sub, rows_per_sub)])

    return k(hs, src2d, dst2d)


def _sc_gather(ew, ew2, i1a2d, i1b2d, i22d):
    nq = i1a2d.shape[0] * i1a2d.shape[1]
    rows_pt = i1a2d.shape[0] // NW
    mesh = plsc.VectorSubcoreMesh(core_axis_name="c", subcore_axis_name="s")

    @functools.partial(
        pl.kernel, mesh=mesh,
        out_type=[jax.ShapeDtypeStruct((nq, E), jnp.float32),
                  jax.ShapeDtypeStruct((nq, E), jnp.float32)],
        scratch_types=[
            pltpu.VMEM((rows_pt, EDGE_COLS), jnp.int32),
            pltpu.VMEM((rows_pt, EDGE_COLS), jnp.int32),
            pltpu.VMEM((rows_pt, EDGE_COLS), jnp.int32),
            pltpu.VMEM((EDGE_COLS, E), jnp.float32),
            pltpu.VMEM((EDGE_COLS, E), jnp.float32),
            pltpu.VMEM((EDGE_COLS, E), jnp.float32),
            pltpu.SemaphoreType.DMA,
        ],
        compiler_params=pltpu.CompilerParams(use_tc_tiling_on_sc=False),
    )
    def k(ew_hbm, ew2_hbm, ia_hbm, ib_hbm, i2_hbm, zq_hbm, zq2_hbm,
          iav, ibv, i2v, r1, r2, r3, sem):
        c = lax.axis_index("c")
        s = lax.axis_index("s")
        wid = c * NS + s
        pltpu.sync_copy(ia_hbm.at[pl.ds(wid * rows_pt, rows_pt)], iav)
        pltpu.sync_copy(ib_hbm.at[pl.ds(wid * rows_pt, rows_pt)], ibv)
        pltpu.sync_copy(i2_hbm.at[pl.ds(wid * rows_pt, rows_pt)], i2v)

        def body(j, _):
            base = (wid * rows_pt + j) * EDGE_COLS
            pltpu.async_copy(ew_hbm.at[iav.at[j]], r1, sem).wait()
            pltpu.async_copy(ew_hbm.at[ibv.at[j]], r2, sem).wait()
            pltpu.async_copy(ew2_hbm.at[i2v.at[j]], r3, sem).wait()

            def row_body(r, _):
                for cc in range(E // 16):
                    sl = pl.ds(cc * 16, 16)
                    r1[r, sl] = (r1[r, sl] + r2[r, sl]) * 0.5
                return 0
            lax.fori_loop(0, EDGE_COLS, row_body, 0)
            pltpu.sync_copy(r1, zq_hbm.at[pl.ds(base, EDGE_COLS)])
            pltpu.sync_copy(r3, zq2_hbm.at[pl.ds(base, EDGE_COLS)])
            return 0
        lax.fori_loop(0, rows_pt, body, 0)

    return k(ew, ew2, i1a2d, i1b2d, i22d)


def _finish_body(nrows, zfall_ref, zq_ref, zq2_ref,
                 zqs_ref, zq2s_ref, loss_ref):
    i = pl.program_id(0)
    zfb = zfall_ref[:, :E]
    zf2b = zfall_ref[:, E:]
    zq = zq_ref[...]
    zq2 = zq2_ref[...]
    zqs_ref[...] = zfb + (zq - zfb)
    zq2s_ref[...] = zf2b + (zq2 - zf2b)
    part = jnp.sum((zq - zfb) ** 2) + jnp.sum((zq2 - zf2b) ** 2)
    contrib = part * ((1.0 + BETA) / (nrows * E))
    prev = jnp.where(i == 0, jnp.zeros((1, 1), jnp.float32), loss_ref[...])
    loss_ref[...] = prev + contrib


def _finish(zfall, zq, zq2):
    nrows = zfall.shape[0]
    blk = 2048
    row = lambda i: (i, 0)
    full = lambda i: (0, 0)
    return pl.pallas_call(
        functools.partial(_finish_body, nrows),
        grid=(nrows // blk,),
        in_specs=[pl.BlockSpec((blk, 2 * E), row),
                  pl.BlockSpec((blk, E), row),
                  pl.BlockSpec((blk, E), row)],
        out_specs=[pl.BlockSpec((blk, E), row),
                   pl.BlockSpec((blk, E), row),
                   pl.BlockSpec((1, 1), full)],
        out_shape=[jax.ShapeDtypeStruct((nrows, E), jnp.float32),
                   jax.ShapeDtypeStruct((nrows, E), jnp.float32),
                   jax.ShapeDtypeStruct((1, 1), jnp.float32)],
    )(zfall, zq, zq2)


def _prep_body(degp_ref, mm1_ref, dinv_ref, hs1_ref):
    deg = jnp.sum(degp_ref[...], axis=0) + 1.0
    dinv = 1.0 / jnp.sqrt(deg)
    dinv_ref[...] = dinv[:, None]
    hs1_ref[...] = mm1_ref[...] * dinv[:, None]


def _prep(degp, mm1):
    return pl.pallas_call(
        _prep_body,
        out_shape=[jax.ShapeDtypeStruct((N, 1), jnp.float32),
                   jax.ShapeDtypeStruct((N, E), jnp.float32)],
    )(degp, mm1)


def _layer2_body(accp_ref, hs1_ref, dinv_ref, b1_ref, w2_ref, hs2_ref):
    dinv = dinv_ref[...]
    h2 = dinv * (accp_ref[0] + accp_ref[1] + hs1_ref[...]) + b1_ref[...]
    h2 = jnp.maximum(h2, 0.0)
    hs2_ref[...] = jnp.dot(h2, w2_ref[...],
                           preferred_element_type=jnp.float32) * dinv


def _layer2(accp1, hs1, dinv, b1_2d, W2):
    return pl.pallas_call(
        _layer2_body,
        out_shape=jax.ShapeDtypeStruct((N, E), jnp.float32),
    )(accp1, hs1, dinv, b1_2d, W2)


def _codebooks_body(accp_ref, hs2_ref, dinv_ref, b2_ref,
                    ew_ref, ew2_ref, ewt_ref, sqall_ref):
    total = (dinv_ref[...] * (accp_ref[0] + accp_ref[1] + hs2_ref[...])
             + b2_ref[...])
    ew = total[:ADJ]
    ew2 = total[ADJ:]
    ew_ref[...] = ew
    ew2_ref[...] = ew2
    zpad = jnp.zeros((E, ADJ), jnp.float32)
    top = jnp.concatenate([ew.T, zpad], axis=1)
    bot = jnp.concatenate([zpad, ew2.T], axis=1)
    ewt_ref[...] = jnp.concatenate([top, bot], axis=0)
    sqall_ref[...] = jnp.concatenate(
        [jnp.sum(ew ** 2, axis=1), jnp.sum(ew2 ** 2, axis=1)])[None, :]


def _codebooks(accp2, hs2, dinv, b2_2d):
    return pl.pallas_call(
        _codebooks_body,
        out_shape=[jax.ShapeDtypeStruct((ADJ, E), jnp.float32),
                   jax.ShapeDtypeStruct((ADJ, E), jnp.float32),
                   jax.ShapeDtypeStruct((2 * E, 2 * ADJ), jnp.float32),
                   jax.ShapeDtypeStruct((1, 2 * ADJ), jnp.float32)],
    )(accp2, hs2, dinv, b2_2d)


def _quant_body(zfall_ref, ewt_ref, sqall_ref, i1a_ref, i1b_ref, i2_ref):
    blk = zfall_ref.shape[0]
    fiota = lax.broadcasted_iota(jnp.int32, (1, ADJ), 1).astype(jnp.float32)
    big = jnp.float32(1e9)
    nn = (((1,), (0,)), ((), ()))

    zfall = zfall_ref[...]
    zfb = zfall[:, :E]
    zf2b = zfall[:, E:]
    mmall = lax.dot_general(zfall, ewt_ref[...], nn,
                            preferred_element_type=jnp.float32)
    sqall = sqall_ref[...]

    d = (jnp.sum(zfb ** 2, axis=1, keepdims=True) + sqall[:, :ADJ]
         - 2.0 * mmall[:, :ADJ])
    m1 = jnp.min(d, axis=1, keepdims=True)
    i1f = jnp.min(jnp.where(d == m1, fiota, big), axis=1)
    d2 = jnp.where(fiota == i1f[:, None], jnp.float32(jnp.inf), d)
    m2 = jnp.min(d2, axis=1, keepdims=True)
    i1bf = jnp.min(jnp.where(d2 == m2, fiota, big), axis=1)
    i1a_ref[...] = i1f[:, None].astype(jnp.int32)
    i1b_ref[...] = i1bf[:, None].astype(jnp.int32)

    dn = (jnp.sum(zf2b ** 2, axis=1, keepdims=True) + sqall[:, ADJ:]
          - 2.0 * mmall[:, ADJ:])
    mn = jnp.min(dn, axis=1, keepdims=True)
    i2f = jnp.min(jnp.where(dn == mn, fiota, big), axis=1)
    i2_ref[...] = i2f[:, None].astype(jnp.int32)


def _quant(zfall, ewt, sqall):
    nrows = zfall.shape[0]
    blk = 512
    grid = (nrows // blk,)
    full = lambda i: (0, 0)
    row = lambda i: (i, 0)
    return pl.pallas_call(
        _quant_body,
        grid=grid,
        in_specs=[pl.BlockSpec((blk, 2 * E), row),
                  pl.BlockSpec((2 * E, 2 * ADJ), full),
                  pl.BlockSpec((1, 2 * ADJ), full)],
        out_specs=[pl.BlockSpec((blk, 1), row),
                   pl.BlockSpec((blk, 1), row),
                   pl.BlockSpec((blk, 1), row)],
        out_shape=[jax.ShapeDtypeStruct((nrows, 1), jnp.int32),
                   jax.ShapeDtypeStruct((nrows, 1), jnp.int32),
                   jax.ShapeDtypeStruct((nrows, 1), jnp.int32)],
    )(zfall, ewt, sqall)


def kernel(z, code, edge_index, W1, b1, W2, b2):
    b = z.shape[0]
    src2d = edge_index[0].reshape(-1, EDGE_COLS)
    dst2d = edge_index[1].reshape(-1, EDGE_COLS)

    degp = _sc_degree(dst2d)
    mm1 = _mm1(code, W1)
    dinv, hs1 = _prep(degp, mm1)
    accp1 = _sc_scatter(hs1, src2d, dst2d)
    hs2 = _layer2(accp1, hs1, dinv, b1.reshape(1, E), W2)
    accp2 = _sc_scatter(hs2, src2d, dst2d)
    ew, ew2, ewt, sqall = _codebooks(accp2, hs2, dinv, b2.reshape(1, E))

    zfall = jnp.transpose(z, (0, 2, 3, 1)).reshape(-1, 2 * E)
    i1a, i1b, i2 = _quant(zfall, ewt, sqall)
    zqr, zq2r = _sc_gather(ew, ew2,
                           i1a.reshape(-1, EDGE_COLS),
                           i1b.reshape(-1, EDGE_COLS),
                           i2.reshape(-1, EDGE_COLS))
    zq, zq2, lossm = _finish(zfall, zqr, zq2r)

    h, w = z.shape[2], z.shape[3]
    z_adj_q = jnp.transpose(zq.reshape(b, h, w, E), (0, 3, 1, 2))
    z_noun_q = jnp.transpose(zq2.reshape(b, h, w, E), (0, 3, 1, 2))
    z_q = jnp.concatenate([z_adj_q, z_noun_q], axis=1)
    idx1 = jnp.concatenate([i1a, i1b], axis=1).reshape(b, -1)
    idx2 = i2.reshape(b, -1)
    loss = lossm.reshape(())
    return z_q, loss, idx1, idx2
